# baseline (speedup 1.0000x reference)
import functools
import jax
import jax.numpy as jnp
from jax.experimental import pallas as pl
from jax.experimental.pallas import tpu as pltpu
from jax.experimental.pallas import tpu_sc as plsc

B = 8
IN_CH = 256
OUT_CH = 512
NV_FINE = 10242
NV_COARSE = 2562
NF = 5120
N_ROWS = B * NV_COARSE
ROW_TILE = 168
N_ROW_TILES = N_ROWS // ROW_TILE
VF_TILE = 512
N_VF_TILES = (NV_FINE + VF_TILE - 1) // VF_TILE
VC_TILE = 128
N_VC_TILES = (NV_COARSE + VC_TILE - 1) // VC_TILE
EPS = 1e-5


def _gelu(x):
    return 0.5 * x * (1.0 + jax.lax.erf(x * 0.7071067811865476))


def _conv_fine_body(x_ref, w_ref, y1_ref, ys_ref):
    xb = x_ref[0]
    y = jax.lax.dot_general(
        xb, w_ref[...], (((0,), (0,)), ((), ())),
        preferred_element_type=jnp.float32)
    y1_ref[0] = y[:, :IN_CH]
    ys_ref[0] = y[:, IN_CH:]


def _conv_fine(x, wcat_t):
    return pl.pallas_call(
        _conv_fine_body,
        grid=(B, N_VF_TILES),
        in_specs=[
            pl.BlockSpec((1, IN_CH, VF_TILE), lambda b, i: (b, 0, i)),
            pl.BlockSpec((IN_CH, IN_CH + OUT_CH), lambda b, i: (0, 0)),
        ],
        out_specs=[
            pl.BlockSpec((1, VF_TILE, IN_CH), lambda b, i: (b, i, 0)),
            pl.BlockSpec((1, VF_TILE, OUT_CH), lambda b, i: (b, i, 0)),
        ],
        out_shape=[
            jax.ShapeDtypeStruct((B, NV_FINE, IN_CH), jnp.float32),
            jax.ShapeDtypeStruct((B, NV_FINE, OUT_CH), jnp.float32),
        ],
    )(x, wcat_t)


def _pool_sc(table, idx_flat, c, n_groups):
    info = plsc.get_sparse_core_info()
    n_workers = info.num_cores * info.num_subcores
    n_iters = (n_groups + n_workers - 1) // n_workers
    mesh = plsc.VectorSubcoreMesh(core_axis_name="c", subcore_axis_name="s")

    @functools.partial(
        pl.kernel, mesh=mesh,
        out_type=jax.ShapeDtypeStruct((n_groups * 8, c), jnp.float32),
        scratch_types=[
            pltpu.VMEM((64,), jnp.int32),
            pltpu.VMEM((64, c), jnp.float32),
            pltpu.VMEM((8, c), jnp.float32),
            pltpu.SemaphoreType.DMA,
        ],
    )
    def k(table_hbm, idx_hbm, out_hbm, idx_v, rows_v, out_v, sem):
        wid = jax.lax.axis_index("s") * info.num_cores + jax.lax.axis_index("c")

        def step(t, carry):
            g = wid + t * n_workers

            @pl.when(g < n_groups)
            def _():
                pltpu.sync_copy(idx_hbm.at[pl.ds(g * 64, 64)], idx_v)
                pltpu.async_copy(table_hbm.at[idx_v], rows_v, sem).wait()

                def chunk(ci, cc):
                    off = ci * 16
                    for gi in range(8):
                        acc = rows_v[gi * 8, pl.ds(off, 16)]
                        for j in range(1, 8):
                            acc = jnp.maximum(acc, rows_v[gi * 8 + j, pl.ds(off, 16)])
                        out_v[gi, pl.ds(off, 16)] = acc
                    return cc

                jax.lax.fori_loop(0, c // 16, chunk, 0)
                pltpu.sync_copy(out_v, out_hbm.at[pl.ds(g * 8, 8)])

            return carry

        jax.lax.fori_loop(0, n_iters, step, 0)

    return k(table, idx_flat)


def _spmm_sc(table, idx_flat, w_list, n_rows, k, g):
    cw = table.shape[1]
    info = plsc.get_sparse_core_info()
    n_workers = info.num_cores * info.num_subcores
    n_groups = n_rows // g
    n_iters = (n_groups + n_workers - 1) // n_workers
    n_out = len(w_list)
    mesh = plsc.VectorSubcoreMesh(core_axis_name="c", subcore_axis_name="s")
    out_t = jax.ShapeDtypeStruct((n_rows, cw), jnp.float32)

    @functools.partial(
        pl.kernel, mesh=mesh,
        out_type=[out_t] * n_out,
        scratch_types=[pltpu.VMEM((g * k,), jnp.int32),
                       pltpu.VMEM((g * k, cw), jnp.float32)]
                      + [pltpu.VMEM((g * k, 16), jnp.float32)] * n_out
                      + [pltpu.VMEM((g, cw), jnp.float32)] * n_out
                      + [pltpu.SemaphoreType.DMA],
    )
    def kern(*refs):
        table_hbm, idx_hbm = refs[0], refs[1]
        w_hbm = refs[2:2 + n_out]
        out_hbm = refs[2 + n_out:2 + 2 * n_out]
        idx_v, rows_v = refs[2 + 2 * n_out], refs[3 + 2 * n_out]
        w_s = refs[4 + 2 * n_out:4 + 2 * n_out + n_out]
        out_v = refs[4 + 3 * n_out:4 + 3 * n_out + n_out]
        sem = refs[-1]
        wid = jax.lax.axis_index("s") * info.num_cores + jax.lax.axis_index("c")

        def step(t, carry):
            gg = wid + t * n_workers

            @pl.when(gg < n_groups)
            def _():
                base = gg * (g * k)
                pltpu.sync_copy(idx_hbm.at[pl.ds(base, g * k)], idx_v)
                for o in range(n_out):
                    pltpu.sync_copy(w_hbm[o].at[pl.ds(base, g * k)], w_s[o])
                pltpu.async_copy(table_hbm.at[idx_v], rows_v, sem).wait()
                wv = [[w_s[o][i] for i in range(g * k)] for o in range(n_out)]

                def chunk(ci, cc):
                    off = ci * 16
                    for gi in range(g):
                        loads = [rows_v[gi * k + j, pl.ds(off, 16)]
                                 for j in range(k)]
                        for o in range(n_out):
                            acc = loads[0] * wv[o][gi * k]
                            for j in range(1, k):
                                acc = acc + loads[j] * wv[o][gi * k + j]
                            out_v[o][gi, pl.ds(off, 16)] = acc
                    return cc

                jax.lax.fori_loop(0, cw // 16, chunk, 0)
                for o in range(n_out):
                    pltpu.sync_copy(out_v[o], out_hbm[o].at[pl.ds(gg * g, g)])

            return carry

        jax.lax.fori_loop(0, n_iters, step, 0)

    res = kern(table, idx_flat, *w_list)
    return list(res) if isinstance(res, (list, tuple)) else [res]


def _pad_fanin(cols, vals_list, k_pad):
    nr, kk = cols.shape
    pc = jnp.concatenate(
        [cols, jnp.zeros((nr, k_pad - kk), jnp.int32)], axis=1).reshape(-1)
    pvs = [jnp.broadcast_to(
        jnp.concatenate([v, jnp.zeros((nr, k_pad - kk), jnp.float32)],
                        axis=1).reshape(-1)[:, None],
        (nr * k_pad, 16))
        for v in vals_list]
    return pc, pvs


def _stats_body(x_ref, o_ref):
    i = pl.program_id(0)

    @pl.when(i == 0)
    def _():
        o_ref[...] = jnp.zeros_like(o_ref)

    r = x_ref[...]
    o_ref[...] += jnp.stack([jnp.sum(r, axis=0), jnp.sum(r * r, axis=0)])


def _stats(rows):
    c = rows.shape[1]
    return pl.pallas_call(
        _stats_body,
        grid=(N_ROW_TILES,),
        in_specs=[pl.BlockSpec((ROW_TILE, c), lambda i: (i, 0))],
        out_specs=pl.BlockSpec((2, c), lambda i: (0, 0)),
        out_shape=jax.ShapeDtypeStruct((2, c), jnp.float32),
    )(rows)


def _scale_off(sums, g, be):
    mean = sums[0] / N_ROWS
    var = sums[1] / N_ROWS - mean * mean
    scale = g * jax.lax.rsqrt(var + EPS)
    return scale, be - mean * scale


def _apply_gelu_body(x_ref, s_ref, g_ref, b_ref, o_ref):
    scale, off = _scale_off(s_ref[...], g_ref[...], b_ref[...])
    o_ref[...] = _gelu(x_ref[...] * scale[None, :] + off[None, :])


def _apply_gelu(rows, sums, g, be):
    c = rows.shape[1]
    return pl.pallas_call(
        _apply_gelu_body,
        grid=(N_ROW_TILES,),
        in_specs=[
            pl.BlockSpec((ROW_TILE, c), lambda i: (i, 0)),
            pl.BlockSpec((2, c), lambda i: (0, 0)),
            pl.BlockSpec((c,), lambda i: (0,)),
            pl.BlockSpec((c,), lambda i: (0,)),
        ],
        out_specs=pl.BlockSpec((ROW_TILE, c), lambda i: (i, 0)),
        out_shape=jax.ShapeDtypeStruct((N_ROWS, c), jnp.float32),
    )(rows, sums, g, be)


def _feat_mm_body(h_ref, lap_ref, gve_ref, gvn_ref, w_ref, o_ref):
    ht = h_ref[...].reshape(VC_TILE * B, IN_CH)
    lt = lap_ref[...].reshape(VC_TILE * B, IN_CH)
    et = gve_ref[...].reshape(VC_TILE * B, IN_CH)
    nt = gvn_ref[...].reshape(VC_TILE * B, IN_CH)
    w = w_ref[...]
    acc = jnp.dot(ht, w[0:IN_CH], preferred_element_type=jnp.float32)
    acc += jnp.dot(lt, w[IN_CH:2 * IN_CH], preferred_element_type=jnp.float32)
    acc += jnp.dot(et, w[2 * IN_CH:3 * IN_CH], preferred_element_type=jnp.float32)
    acc += jnp.dot(nt, w[3 * IN_CH:4 * IN_CH], preferred_element_type=jnp.float32)
    o_ref[...] = acc.reshape(VC_TILE, B, IN_CH)


def _feat_mm(h3, lap3, gve3, gvn3, wm):
    spec = pl.BlockSpec((VC_TILE, B, IN_CH), lambda i: (i, 0, 0))
    return pl.pallas_call(
        _feat_mm_body,
        grid=(N_VC_TILES,),
        in_specs=[spec, spec, spec, spec,
                  pl.BlockSpec((4 * IN_CH, IN_CH), lambda i: (0, 0))],
        out_specs=spec,
        out_shape=jax.ShapeDtypeStruct((NV_COARSE, B, IN_CH), jnp.float32),
    )(h3, lap3, gve3, gvn3, wm)


def _w3_body(m_ref, s_ref, g_ref, b_ref, w_ref, o_ref):
    scale, off = _scale_off(s_ref[...], g_ref[...], b_ref[...])
    z = _gelu(m_ref[...] * scale[None, None, :] + off[None, None, :])
    t = jnp.dot(z.reshape(VC_TILE * B, IN_CH), w_ref[...],
                preferred_element_type=jnp.float32)
    o_ref[...] = jnp.transpose(t.reshape(VC_TILE, B, OUT_CH), (1, 0, 2))


def _w3(m3, sums2, g2, be2, w3t):
    return pl.pallas_call(
        _w3_body,
        grid=(N_VC_TILES,),
        in_specs=[
            pl.BlockSpec((VC_TILE, B, IN_CH), lambda i: (i, 0, 0)),
            pl.BlockSpec((2, IN_CH), lambda i: (0, 0)),
            pl.BlockSpec((IN_CH,), lambda i: (0,)),
            pl.BlockSpec((IN_CH,), lambda i: (0,)),
            pl.BlockSpec((IN_CH, OUT_CH), lambda i: (0, 0)),
        ],
        out_specs=pl.BlockSpec((B, VC_TILE, OUT_CH), lambda i: (0, i, 0)),
        out_shape=jax.ShapeDtypeStruct((B, NV_COARSE, OUT_CH), jnp.float32),
    )(m3, sums2, g2, be2, w3t)


def _final_body(t_ref, p_ref, s3_ref, g3_ref, b3_ref, ss_ref, gs_ref, bs_ref, o_ref):
    sc3, of3 = _scale_off(s3_ref[...], g3_ref[...], b3_ref[...])
    scs, ofs = _scale_off(ss_ref[...], gs_ref[...], bs_ref[...])
    r = (t_ref[0] * sc3[None, :] + of3[None, :]
         + p_ref[0] * scs[None, :] + ofs[None, :])
    r = _gelu(r)
    o_ref[0] = jnp.transpose(r, (1, 0))


def _final(t, ps, sums3, g3, be3, sums_s, gs, bes):
    vec = pl.BlockSpec((OUT_CH,), lambda b, i: (0,))
    st = pl.BlockSpec((2, OUT_CH), lambda b, i: (0, 0))
    blk = pl.BlockSpec((1, VC_TILE, OUT_CH), lambda b, i: (b, i, 0))
    return pl.pallas_call(
        _final_body,
        grid=(B, N_VC_TILES),
        in_specs=[blk, blk, st, vec, vec, st, vec, vec],
        out_specs=pl.BlockSpec((1, OUT_CH, VC_TILE), lambda b, i: (b, 0, i)),
        out_shape=jax.ShapeDtypeStruct((B, OUT_CH, NV_COARSE), jnp.float32),
    )(t, ps, sums3, g3, be3, sums_s, gs, bes)


def kernel(x, W1, b1, g1, be1, coeffs, mcb, g2, be2, W3, b3, g3, be3,
           Ws, bs, gs, bes, g_rows, g_cols, g_vals, l_rows, l_cols, l_vals,
           f_rows, f_cols, f_vals, ns, ew, vert_idx, patches):
    wcat_t = jnp.concatenate([W1, Ws], axis=0).T
    wm = jnp.transpose(coeffs, (2, 1, 0)).reshape(4 * IN_CH, IN_CH)
    w3t = W3.T
    pidx = vert_idx[patches].astype(jnp.int32)
    gc9 = jnp.transpose(g_cols.astype(jnp.int32).reshape(3, NF, 3),
                        (1, 0, 2)).reshape(NF, 9)
    gv9 = jnp.transpose(g_vals.reshape(3, NF, 3), (1, 0, 2)).reshape(NF, 3, 3)
    we9 = (ew[:, :, None] * gv9).reshape(NF, 9)
    wn9 = (ns[:, :, None] * gv9).reshape(NF, 9)
    lc7 = l_cols.astype(jnp.int32).reshape(NV_COARSE, 7)
    lv7 = l_vals.reshape(NV_COARSE, 7)
    fc6 = f_cols.astype(jnp.int32).reshape(NV_COARSE, 6)
    fv6 = f_vals.reshape(NV_COARSE, 6)

    pv8 = jnp.concatenate([pidx, pidx[:, :1]], axis=1)
    boff = jnp.arange(B, dtype=jnp.int32) * NV_FINE
    pidx_h = (boff[None, :, None] + pv8[:, None, :]).reshape(-1)
    pidx_s = (boff[:, None, None] + pv8[None, :, :]).reshape(-1)

    y1, ys = _conv_fine(x, wcat_t)

    h_rows = _pool_sc(y1.reshape(B * NV_FINE, IN_CH), pidx_h,
                      IN_CH, NV_COARSE)
    ps_rows = _pool_sc(ys.reshape(B * NV_FINE, OUT_CH), pidx_s,
                       OUT_CH, NV_COARSE)
    ps = ps_rows.reshape(B, NV_COARSE, OUT_CH)

    sums1 = _stats(h_rows)
    h = _apply_gelu(h_rows, sums1, g1, be1)
    h3 = h.reshape(NV_COARSE, B, IN_CH)
    hv = h.reshape(NV_COARSE, B * IN_CH)

    nvc_pad = NV_COARSE + 2
    rp = lambda a: jnp.concatenate(
        [a, jnp.zeros((nvc_pad - NV_COARSE,) + a.shape[1:], a.dtype)], axis=0)
    gc_f, (we_f, wn_f) = _pad_fanin(gc9, [we9, wn9], 12)
    lc_f, (lv_f,) = _pad_fanin(rp(lc7), [rp(lv7)], 8)
    fc_f, (fv_f,) = _pad_fanin(rp(fc6), [rp(fv6)], 8)

    gfe, gfn = _spmm_sc(hv, gc_f, [we_f, wn_f], NF, 12, 2)
    lap = _spmm_sc(hv, lc_f, [lv_f], nvc_pad, 8, 4)[0]
    gve = _spmm_sc(gfe, fc_f, [fv_f], nvc_pad, 8, 4)[0]
    gvn = _spmm_sc(gfn, fc_f, [fv_f], nvc_pad, 8, 4)[0]
    lap = lap[:NV_COARSE].reshape(NV_COARSE, B, IN_CH)
    gve = gve[:NV_COARSE].reshape(NV_COARSE, B, IN_CH)
    gvn = gvn[:NV_COARSE].reshape(NV_COARSE, B, IN_CH)

    m3 = _feat_mm(h3, lap, gve, gvn, wm)

    sums2 = _stats(m3.reshape(N_ROWS, IN_CH))
    t = _w3(m3, sums2, g2, be2, w3t)

    sums3 = _stats(t.reshape(N_ROWS, OUT_CH))
    sums_s = _stats(ps.reshape(N_ROWS, OUT_CH))
    return _final(t, ps, sums3, g3, be3, sums_s, gs, bes)


# speedup vs baseline: 1.1361x; 1.1361x over previous
import functools
import jax
import jax.numpy as jnp
from jax.experimental import pallas as pl
from jax.experimental.pallas import tpu as pltpu
from jax.experimental.pallas import tpu_sc as plsc

B = 8
IN_CH = 256
OUT_CH = 512
NV_FINE = 10242
NV_COARSE = 2562
NF = 5120
VF_PAD = 10752
ZROW = 10242
VPAD = 2688
NPAD = B * VPAD
N_ROWS = B * NV_COARSE
ROW_TILE = 168
N_STAT_TILES = N_ROWS // ROW_TILE
N_APPLY_TILES = NPAD // ROW_TILE
VF_TILE = 512
N_VF_TILES = VF_PAD // VF_TILE
VC_TILE = 128
N_VC_TILES = VPAD // VC_TILE
EPS = 1e-5

_SC_INFO = plsc.get_sparse_core_info()
NWORK = _SC_INFO.num_cores * _SC_INFO.num_subcores


def _gelu(x):
    return 0.5 * x * (1.0 + jax.lax.erf(x * 0.7071067811865476))


def _wid():
    return (jax.lax.axis_index("s") * _SC_INFO.num_cores
            + jax.lax.axis_index("c"))


def _conv_fine_body(x_ref, w_ref, y1_ref, ys_ref):
    i = pl.program_id(1)
    xb = x_ref[0]
    y = jax.lax.dot_general(
        xb, w_ref[...], (((0,), (0,)), ((), ())),
        preferred_element_type=jnp.float32)
    rows = jax.lax.broadcasted_iota(jnp.int32, (VF_TILE, 1), 0)
    y = jnp.where(rows < NV_FINE - i * VF_TILE, y, 0.0)
    y1_ref[0] = y[:, :IN_CH]
    ys_ref[0] = y[:, IN_CH:]


def _conv_fine(x, wcat_t):
    return pl.pallas_call(
        _conv_fine_body,
        grid=(B, N_VF_TILES),
        in_specs=[
            pl.BlockSpec((1, IN_CH, VF_TILE), lambda b, i: (b, 0, i)),
            pl.BlockSpec((IN_CH, IN_CH + OUT_CH), lambda b, i: (0, 0)),
        ],
        out_specs=[
            pl.BlockSpec((1, VF_TILE, IN_CH), lambda b, i: (b, i, 0)),
            pl.BlockSpec((1, VF_TILE, OUT_CH), lambda b, i: (b, i, 0)),
        ],
        out_shape=[
            jax.ShapeDtypeStruct((B, VF_PAD, IN_CH), jnp.float32),
            jax.ShapeDtypeStruct((B, VF_PAD, OUT_CH), jnp.float32),
        ],
    )(x, wcat_t)


def _pool_sc(table, idx_flat, c):
    n_groups = VPAD
    per_w = n_groups // NWORK
    mesh = plsc.VectorSubcoreMesh(core_axis_name="c", subcore_axis_name="s")

    @functools.partial(
        pl.kernel, mesh=mesh,
        out_type=jax.ShapeDtypeStruct((n_groups * 8, c), jnp.float32),
        scratch_types=[
            pltpu.VMEM((per_w * 64,), jnp.int32),
            pltpu.VMEM((64, c), jnp.float32),
            pltpu.VMEM((64, c), jnp.float32),
            pltpu.VMEM((8, c), jnp.float32),
            pltpu.VMEM((8, c), jnp.float32),
            pltpu.SemaphoreType.DMA,
            pltpu.SemaphoreType.DMA,
            pltpu.SemaphoreType.DMA,
            pltpu.SemaphoreType.DMA,
        ],
    )
    def k(table_hbm, idx_hbm, out_hbm, idx_all, r0, r1, o0, o1,
          sr0, sr1, so0, so1):
        rows_v = [r0, r1]
        out_v = [o0, o1]
        sem_r = [sr0, sr1]
        sem_o = [so0, so1]
        base_g = _wid() * per_w
        pltpu.sync_copy(idx_hbm.at[pl.ds(base_g * 64, per_w * 64)], idx_all)

        def gather(t, b):
            return pltpu.async_copy(
                table_hbm.at[idx_all.at[pl.ds(t * 64, 64)]],
                rows_v[b], sem_r[b])

        def outcopy(t, b):
            return pltpu.async_copy(
                out_v[b], out_hbm.at[pl.ds((base_g + t) * 8, 8)], sem_o[b])

        gather(0, 0)

        def sstep(s, carry):
            for b in (0, 1):
                t = s * 2 + b

                @pl.when(t + 1 < per_w)
                def _():
                    gather(t + 1, 1 - b)

                pltpu.make_async_copy(
                    table_hbm.at[idx_all.at[pl.ds(t * 64, 64)]],
                    rows_v[b], sem_r[b]).wait()

                @pl.when(t >= 2)
                def _():
                    pltpu.make_async_copy(
                        out_v[b], out_hbm.at[pl.ds((base_g + t - 2) * 8, 8)],
                        sem_o[b]).wait()

                def chunk(ci, cc):
                    off = ci * 16
                    for gi in range(8):
                        acc = rows_v[b][gi * 8, pl.ds(off, 16)]
                        for j in range(1, 8):
                            acc = jnp.maximum(
                                acc, rows_v[b][gi * 8 + j, pl.ds(off, 16)])
                        out_v[b][gi, pl.ds(off, 16)] = acc
                    return cc

                jax.lax.fori_loop(0, c // 16, chunk, 0)
                outcopy(t, b)
            return carry

        jax.lax.fori_loop(0, per_w // 2, sstep, 0)
        for b in (0, 1):
            pltpu.make_async_copy(
                out_v[b], out_hbm.at[pl.ds((base_g + per_w - 2 + b) * 8, 8)],
                sem_o[b]).wait()

    return k(table, idx_flat)


def _spmm_sc(table, idx_flat, w_list, n_rows, k, g):
    cw = table.shape[1]
    gk = g * k
    n_groups = n_rows // g
    per_w = n_groups // NWORK
    n_out = len(w_list)
    mesh = plsc.VectorSubcoreMesh(core_axis_name="c", subcore_axis_name="s")
    out_t = jax.ShapeDtypeStruct((n_rows, cw), jnp.float32)

    @functools.partial(
        pl.kernel, mesh=mesh,
        out_type=[out_t] * n_out,
        scratch_types=[pltpu.VMEM((per_w * gk,), jnp.int32)]
                      + [pltpu.VMEM((gk, cw), jnp.float32)] * 2
                      + [pltpu.VMEM((gk, 16), jnp.float32)] * (2 * n_out)
                      + [pltpu.VMEM((g, cw), jnp.float32)] * (2 * n_out)
                      + [pltpu.SemaphoreType.DMA] * 6,
    )
    def kern(*refs):
        table_hbm, idx_hbm = refs[0], refs[1]
        w_hbm = refs[2:2 + n_out]
        out_hbm = refs[2 + n_out:2 + 2 * n_out]
        sc = list(refs[2 + 2 * n_out:])
        idx_all = sc[0]
        rows_v = sc[1:3]
        w_v = [sc[3 + 2 * o:5 + 2 * o] for o in range(n_out)]
        out_v = [sc[3 + 2 * n_out + 2 * o:5 + 2 * n_out + 2 * o]
                 for o in range(n_out)]
        sem_r = sc[3 + 4 * n_out:5 + 4 * n_out]
        sem_w = sc[5 + 4 * n_out:7 + 4 * n_out]
        sem_o = sc[7 + 4 * n_out:9 + 4 * n_out]
        base_g = _wid() * per_w
        pltpu.sync_copy(idx_hbm.at[pl.ds(base_g * gk, per_w * gk)], idx_all)

        def issue(t, b):
            pltpu.async_copy(
                table_hbm.at[idx_all.at[pl.ds(t * gk, gk)]],
                rows_v[b], sem_r[b])
            for o in range(n_out):
                pltpu.async_copy(
                    w_hbm[o].at[pl.ds((base_g + t) * gk, gk)],
                    w_v[o][b], sem_w[b])

        def wait_in(t, b):
            pltpu.make_async_copy(
                table_hbm.at[idx_all.at[pl.ds(t * gk, gk)]],
                rows_v[b], sem_r[b]).wait()
            for o in range(n_out):
                pltpu.make_async_copy(
                    w_hbm[o].at[pl.ds((base_g + t) * gk, gk)],
                    w_v[o][b], sem_w[b]).wait()

        def wait_out(t, b):
            for o in range(n_out):
                pltpu.make_async_copy(
                    out_v[o][b], out_hbm[o].at[pl.ds((base_g + t) * g, g)],
                    sem_o[b]).wait()

        issue(0, 0)

        def sstep(s, carry):
            for b in (0, 1):
                t = s * 2 + b

                @pl.when(t + 1 < per_w)
                def _():
                    issue(t + 1, 1 - b)

                wait_in(t, b)

                @pl.when(t >= 2)
                def _():
                    wait_out(t - 2, b)

                wv = [[w_v[o][b][i] for i in range(gk)] for o in range(n_out)]

                def chunk(ci, cc):
                    off = ci * 16
                    for gi in range(g):
                        loads = [rows_v[b][gi * k + j, pl.ds(off, 16)]
                                 for j in range(k)]
                        for o in range(n_out):
                            acc = loads[0] * wv[o][gi * k]
                            for j in range(1, k):
                                acc = acc + loads[j] * wv[o][gi * k + j]
                            out_v[o][b][gi, pl.ds(off, 16)] = acc
                    return cc

                jax.lax.fori_loop(0, cw // 16, chunk, 0)
                for o in range(n_out):
                    pltpu.async_copy(
                        out_v[o][b], out_hbm[o].at[pl.ds((base_g + t) * g, g)],
                        sem_o[b])
            return carry

        jax.lax.fori_loop(0, per_w // 2, sstep, 0)
        for b in (0, 1):
            wait_out(per_w - 2 + b, b)

    res = kern(table, idx_flat, *w_list)
    return list(res) if isinstance(res, (list, tuple)) else [res]


def _pad_fanin(cols, vals_list, k_pad):
    nr, kk = cols.shape
    pc = jnp.concatenate(
        [cols, jnp.zeros((nr, k_pad - kk), jnp.int32)], axis=1).reshape(-1)
    pvs = [jnp.broadcast_to(
        jnp.concatenate([v, jnp.zeros((nr, k_pad - kk), jnp.float32)],
                        axis=1).reshape(-1)[:, None],
        (nr * k_pad, 16))
        for v in vals_list]
    return pc, pvs


def _stats_body(x_ref, o_ref):
    i = pl.program_id(0)

    @pl.when(i == 0)
    def _():
        o_ref[...] = jnp.zeros_like(o_ref)

    r = x_ref[...]
    o_ref[...] += jnp.stack([jnp.sum(r, axis=0), jnp.sum(r * r, axis=0)])


def _stats(rows):
    c = rows.shape[1]
    return pl.pallas_call(
        _stats_body,
        grid=(N_STAT_TILES,),
        in_specs=[pl.BlockSpec((ROW_TILE, c), lambda i: (i, 0))],
        out_specs=pl.BlockSpec((2, c), lambda i: (0, 0)),
        out_shape=jax.ShapeDtypeStruct((2, c), jnp.float32),
    )(rows)


def _stats3d_body(x_ref, o_ref):
    b = pl.program_id(0)
    j = pl.program_id(1)

    @pl.when((b == 0) & (j == 0))
    def _():
        o_ref[...] = jnp.zeros_like(o_ref)

    r = x_ref[0]
    rows = jax.lax.broadcasted_iota(jnp.int32, (VC_TILE, 1), 0)
    r = jnp.where(rows < NV_COARSE - j * VC_TILE, r, 0.0)
    o_ref[...] += jnp.stack([jnp.sum(r, axis=0), jnp.sum(r * r, axis=0)])


def _stats3d(arr):
    c = arr.shape[2]
    return pl.pallas_call(
        _stats3d_body,
        grid=(B, N_VC_TILES),
        in_specs=[pl.BlockSpec((1, VC_TILE, c), lambda b, j: (b, j, 0))],
        out_specs=pl.BlockSpec((2, c), lambda b, j: (0, 0)),
        out_shape=jax.ShapeDtypeStruct((2, c), jnp.float32),
    )(arr)


def _scale_off(sums, g, be):
    mean = sums[0] / N_ROWS
    var = sums[1] / N_ROWS - mean * mean
    scale = g * jax.lax.rsqrt(var + EPS)
    return scale, be - mean * scale


def _apply_gelu_body(x_ref, s_ref, g_ref, b_ref, o_ref):
    scale, off = _scale_off(s_ref[...], g_ref[...], b_ref[...])
    o_ref[...] = _gelu(x_ref[...] * scale[None, :] + off[None, :])


def _apply_gelu(rows, sums, g, be):
    c = rows.shape[1]
    return pl.pallas_call(
        _apply_gelu_body,
        grid=(N_APPLY_TILES,),
        in_specs=[
            pl.BlockSpec((ROW_TILE, c), lambda i: (i, 0)),
            pl.BlockSpec((2, c), lambda i: (0, 0)),
            pl.BlockSpec((c,), lambda i: (0,)),
            pl.BlockSpec((c,), lambda i: (0,)),
        ],
        out_specs=pl.BlockSpec((ROW_TILE, c), lambda i: (i, 0)),
        out_shape=jax.ShapeDtypeStruct((NPAD, c), jnp.float32),
    )(rows, sums, g, be)


def _feat_mm_body(h_ref, lap_ref, gve_ref, gvn_ref, w_ref, o_ref):
    ht = h_ref[...].reshape(VC_TILE * B, IN_CH)
    lt = lap_ref[...].reshape(VC_TILE * B, IN_CH)
    et = gve_ref[...].reshape(VC_TILE * B, IN_CH)
    nt = gvn_ref[...].reshape(VC_TILE * B, IN_CH)
    w = w_ref[...]
    acc = jnp.dot(ht, w[0:IN_CH], preferred_element_type=jnp.float32)
    acc += jnp.dot(lt, w[IN_CH:2 * IN_CH], preferred_element_type=jnp.float32)
    acc += jnp.dot(et, w[2 * IN_CH:3 * IN_CH], preferred_element_type=jnp.float32)
    acc += jnp.dot(nt, w[3 * IN_CH:4 * IN_CH], preferred_element_type=jnp.float32)
    o_ref[...] = acc.reshape(VC_TILE, B, IN_CH)


def _feat_mm(h3, lap3, gve3, gvn3, wm):
    spec = pl.BlockSpec((VC_TILE, B, IN_CH), lambda i: (i, 0, 0))
    return pl.pallas_call(
        _feat_mm_body,
        grid=(N_VC_TILES,),
        in_specs=[spec, spec, spec, spec,
                  pl.BlockSpec((4 * IN_CH, IN_CH), lambda i: (0, 0))],
        out_specs=spec,
        out_shape=jax.ShapeDtypeStruct((VPAD, B, IN_CH), jnp.float32),
    )(h3, lap3, gve3, gvn3, wm)


def _w3_body(m_ref, s_ref, g_ref, b_ref, w_ref, o_ref):
    scale, off = _scale_off(s_ref[...], g_ref[...], b_ref[...])
    z = _gelu(m_ref[...] * scale[None, None, :] + off[None, None, :])
    t = jnp.dot(z.reshape(VC_TILE * B, IN_CH), w_ref[...],
                preferred_element_type=jnp.float32)
    o_ref[...] = jnp.transpose(t.reshape(VC_TILE, B, OUT_CH), (1, 0, 2))


def _w3(m3, sums2, g2, be2, w3t):
    return pl.pallas_call(
        _w3_body,
        grid=(N_VC_TILES,),
        in_specs=[
            pl.BlockSpec((VC_TILE, B, IN_CH), lambda i: (i, 0, 0)),
            pl.BlockSpec((2, IN_CH), lambda i: (0, 0)),
            pl.BlockSpec((IN_CH,), lambda i: (0,)),
            pl.BlockSpec((IN_CH,), lambda i: (0,)),
            pl.BlockSpec((IN_CH, OUT_CH), lambda i: (0, 0)),
        ],
        out_specs=pl.BlockSpec((B, VC_TILE, OUT_CH), lambda i: (0, i, 0)),
        out_shape=jax.ShapeDtypeStruct((B, VPAD, OUT_CH), jnp.float32),
    )(m3, sums2, g2, be2, w3t)


def _final_body(t_ref, p_ref, s3_ref, g3_ref, b3_ref, ss_ref, gs_ref, bs_ref,
                o_ref):
    sc3, of3 = _scale_off(s3_ref[...], g3_ref[...], b3_ref[...])
    scs, ofs = _scale_off(ss_ref[...], gs_ref[...], bs_ref[...])
    r = (t_ref[0] * sc3[None, :] + of3[None, :]
         + p_ref[0] * scs[None, :] + ofs[None, :])
    r = _gelu(r)
    o_ref[0] = jnp.transpose(r, (1, 0))


def _final(t, ps, sums3, g3, be3, sums_s, gs, bes):
    vec = pl.BlockSpec((OUT_CH,), lambda b, i: (0,))
    st = pl.BlockSpec((2, OUT_CH), lambda b, i: (0, 0))
    blk = pl.BlockSpec((1, VC_TILE, OUT_CH), lambda b, i: (b, i, 0))
    return pl.pallas_call(
        _final_body,
        grid=(B, N_VC_TILES),
        in_specs=[blk, blk, st, vec, vec, st, vec, vec],
        out_specs=pl.BlockSpec((1, OUT_CH, VC_TILE), lambda b, i: (b, 0, i)),
        out_shape=jax.ShapeDtypeStruct((B, OUT_CH, NV_COARSE), jnp.float32),
    )(t, ps, sums3, g3, be3, sums_s, gs, bes)


def kernel(x, W1, b1, g1, be1, coeffs, mcb, g2, be2, W3, b3, g3, be3,
           Ws, bs, gs, bes, g_rows, g_cols, g_vals, l_rows, l_cols, l_vals,
           f_rows, f_cols, f_vals, ns, ew, vert_idx, patches):
    wcat_t = jnp.concatenate([W1, Ws], axis=0).T
    wm = jnp.transpose(coeffs, (2, 1, 0)).reshape(4 * IN_CH, IN_CH)
    w3t = W3.T
    pidx = vert_idx[patches].astype(jnp.int32)
    gc9 = jnp.transpose(g_cols.astype(jnp.int32).reshape(3, NF, 3),
                        (1, 0, 2)).reshape(NF, 9)
    gv9 = jnp.transpose(g_vals.reshape(3, NF, 3), (1, 0, 2)).reshape(NF, 3, 3)
    we9 = (ew[:, :, None] * gv9).reshape(NF, 9)
    wn9 = (ns[:, :, None] * gv9).reshape(NF, 9)
    lc7 = l_cols.astype(jnp.int32).reshape(NV_COARSE, 7)
    lv7 = l_vals.reshape(NV_COARSE, 7)
    fc6 = f_cols.astype(jnp.int32).reshape(NV_COARSE, 6)
    fv6 = f_vals.reshape(NV_COARSE, 6)

    pv8 = jnp.concatenate([pidx, pidx[:, :1]], axis=1)
    pv8 = jnp.concatenate(
        [pv8, jnp.full((VPAD - NV_COARSE, 8), ZROW, jnp.int32)], axis=0)
    boff = jnp.arange(B, dtype=jnp.int32) * VF_PAD
    pidx_h = (boff[None, :, None] + pv8[:, None, :]).reshape(-1)
    pidx_s = (boff[:, None, None] + pv8[None, :, :]).reshape(-1)

    rp = lambda a: jnp.concatenate(
        [a, jnp.zeros((VPAD - NV_COARSE,) + a.shape[1:], a.dtype)], axis=0)
    gc_f, (we_f, wn_f) = _pad_fanin(gc9, [we9, wn9], 12)
    lc_f, (lv_f,) = _pad_fanin(rp(lc7), [rp(lv7)], 8)
    fc_f, (fv_f,) = _pad_fanin(rp(fc6), [rp(fv6)], 8)

    y1, ys = _conv_fine(x, wcat_t)

    h_rows = _pool_sc(y1.reshape(B * VF_PAD, IN_CH), pidx_h, IN_CH)
    ps_rows = _pool_sc(ys.reshape(B * VF_PAD, OUT_CH), pidx_s, OUT_CH)
    ps = ps_rows.reshape(B, VPAD, OUT_CH)

    sums1 = _stats(h_rows)
    h = _apply_gelu(h_rows, sums1, g1, be1)
    h3 = h.reshape(VPAD, B, IN_CH)
    hv = h.reshape(VPAD, B * IN_CH)

    gfe, gfn = _spmm_sc(hv, gc_f, [we_f, wn_f], NF, 12, 2)
    lap = _spmm_sc(hv, lc_f, [lv_f], VPAD, 8, 2)[0]
    gve = _spmm_sc(gfe, fc_f, [fv_f], VPAD, 8, 2)[0]
    gvn = _spmm_sc(gfn, fc_f, [fv_f], VPAD, 8, 2)[0]
    lap = lap.reshape(VPAD, B, IN_CH)
    gve = gve.reshape(VPAD, B, IN_CH)
    gvn = gvn.reshape(VPAD, B, IN_CH)

    m3 = _feat_mm(h3, lap, gve, gvn, wm)

    sums2 = _stats(m3.reshape(NPAD, IN_CH))
    t = _w3(m3, sums2, g2, be2, w3t)

    sums3 = _stats3d(t)
    sums_s = _stats3d(ps)
    return _final(t, ps, sums3, g3, be3, sums_s, gs, bes)


# speedup vs baseline: 1.1581x; 1.0194x over previous
import functools
import jax
import jax.numpy as jnp
from jax.experimental import pallas as pl
from jax.experimental.pallas import tpu as pltpu
from jax.experimental.pallas import tpu_sc as plsc

B = 8
IN_CH = 256
OUT_CH = 512
NV_FINE = 10242
NV_COARSE = 2562
NF = 5120
VF_PAD = 10752
ZROW = 10242
VPAD = 2688
NPAD = B * VPAD
N_ROWS = B * NV_COARSE
ROW_TILE = 168
N_STAT_TILES = N_ROWS // ROW_TILE
N_APPLY_TILES = NPAD // ROW_TILE
VF_TILE = 512
N_VF_TILES = VF_PAD // VF_TILE
VC_TILE = 128
N_VC_TILES = VPAD // VC_TILE
EPS = 1e-5

_SC_INFO = plsc.get_sparse_core_info()
NWORK = _SC_INFO.num_cores * _SC_INFO.num_subcores


def _gelu(x):
    return 0.5 * x * (1.0 + jax.lax.erf(x * 0.7071067811865476))


def _wid():
    return (jax.lax.axis_index("s") * _SC_INFO.num_cores
            + jax.lax.axis_index("c"))


def _conv_fine_body(x_ref, w_ref, y1_ref, ys_ref):
    i = pl.program_id(1)
    xb = x_ref[0]
    y = jax.lax.dot_general(
        xb, w_ref[...], (((0,), (0,)), ((), ())),
        preferred_element_type=jnp.float32)
    rows = jax.lax.broadcasted_iota(jnp.int32, (VF_TILE, 1), 0)
    y = jnp.where(rows < NV_FINE - i * VF_TILE, y, 0.0)
    y1_ref[0] = y[:, :IN_CH]
    ys_ref[0] = y[:, IN_CH:]


def _conv_fine(x, wcat_t):
    return pl.pallas_call(
        _conv_fine_body,
        grid=(B, N_VF_TILES),
        in_specs=[
            pl.BlockSpec((1, IN_CH, VF_TILE), lambda b, i: (b, 0, i)),
            pl.BlockSpec((IN_CH, IN_CH + OUT_CH), lambda b, i: (0, 0)),
        ],
        out_specs=[
            pl.BlockSpec((1, VF_TILE, IN_CH), lambda b, i: (b, i, 0)),
            pl.BlockSpec((1, VF_TILE, OUT_CH), lambda b, i: (b, i, 0)),
        ],
        out_shape=[
            jax.ShapeDtypeStruct((B, VF_PAD, IN_CH), jnp.float32),
            jax.ShapeDtypeStruct((B, VF_PAD, OUT_CH), jnp.float32),
        ],
    )(x, wcat_t)


def _pool_sc(table, idx_flat, c):
    n_groups = VPAD
    per_w = n_groups // NWORK
    mesh = plsc.VectorSubcoreMesh(core_axis_name="c", subcore_axis_name="s")

    @functools.partial(
        pl.kernel, mesh=mesh,
        out_type=jax.ShapeDtypeStruct((n_groups * 8, c), jnp.float32),
        scratch_types=[
            pltpu.VMEM((per_w * 64,), jnp.int32),
            pltpu.VMEM((64, c), jnp.float32),
            pltpu.VMEM((64, c), jnp.float32),
            pltpu.VMEM((8, c), jnp.float32),
            pltpu.VMEM((8, c), jnp.float32),
            pltpu.SemaphoreType.DMA,
            pltpu.SemaphoreType.DMA,
            pltpu.SemaphoreType.DMA,
            pltpu.SemaphoreType.DMA,
        ],
    )
    def k(table_hbm, idx_hbm, out_hbm, idx_all, r0, r1, o0, o1,
          sr0, sr1, so0, so1):
        rows_v = [r0, r1]
        out_v = [o0, o1]
        sem_r = [sr0, sr1]
        sem_o = [so0, so1]
        base_g = _wid() * per_w
        pltpu.sync_copy(idx_hbm.at[pl.ds(base_g * 64, per_w * 64)], idx_all)

        def gather(t, b):
            return pltpu.async_copy(
                table_hbm.at[idx_all.at[pl.ds(t * 64, 64)]],
                rows_v[b], sem_r[b])

        def outcopy(t, b):
            return pltpu.async_copy(
                out_v[b], out_hbm.at[pl.ds((base_g + t) * 8, 8)], sem_o[b])

        gather(0, 0)

        def sstep(s, carry):
            for b in (0, 1):
                t = s * 2 + b

                @pl.when(t + 1 < per_w)
                def _():
                    gather(t + 1, 1 - b)

                pltpu.make_async_copy(
                    table_hbm.at[idx_all.at[pl.ds(t * 64, 64)]],
                    rows_v[b], sem_r[b]).wait()

                @pl.when(t >= 2)
                def _():
                    pltpu.make_async_copy(
                        out_v[b], out_hbm.at[pl.ds((base_g + t - 2) * 8, 8)],
                        sem_o[b]).wait()

                def chunk(ci, cc):
                    for u in range(4):
                        off = ci * 64 + u * 16
                        for gi in range(8):
                            acc = rows_v[b][gi * 8, pl.ds(off, 16)]
                            for j in range(1, 7):
                                acc = jnp.maximum(
                                    acc, rows_v[b][gi * 8 + j, pl.ds(off, 16)])
                            out_v[b][gi, pl.ds(off, 16)] = acc
                    return cc

                jax.lax.fori_loop(0, c // 64, chunk, 0)
                outcopy(t, b)
            return carry

        jax.lax.fori_loop(0, per_w // 2, sstep, 0)
        for b in (0, 1):
            pltpu.make_async_copy(
                out_v[b], out_hbm.at[pl.ds((base_g + per_w - 2 + b) * 8, 8)],
                sem_o[b]).wait()

    return k(table, idx_flat)


def _spmm_sc(table, idx_flat, w_list, n_rows, k, g, k_valid):
    cw = table.shape[1]
    gk = g * k
    n_groups = n_rows // g
    per_w = n_groups // NWORK
    n_out = len(w_list)
    mesh = plsc.VectorSubcoreMesh(core_axis_name="c", subcore_axis_name="s")
    out_t = jax.ShapeDtypeStruct((n_rows, cw), jnp.float32)

    @functools.partial(
        pl.kernel, mesh=mesh,
        out_type=[out_t] * n_out,
        scratch_types=[pltpu.VMEM((per_w * gk,), jnp.int32)]
                      + [pltpu.VMEM((gk, cw), jnp.float32)] * 2
                      + [pltpu.VMEM((gk, 16), jnp.float32)] * (2 * n_out)
                      + [pltpu.VMEM((g, cw), jnp.float32)] * (2 * n_out)
                      + [pltpu.SemaphoreType.DMA] * 6,
    )
    def kern(*refs):
        table_hbm, idx_hbm = refs[0], refs[1]
        w_hbm = refs[2:2 + n_out]
        out_hbm = refs[2 + n_out:2 + 2 * n_out]
        sc = list(refs[2 + 2 * n_out:])
        idx_all = sc[0]
        rows_v = sc[1:3]
        w_v = [sc[3 + 2 * o:5 + 2 * o] for o in range(n_out)]
        out_v = [sc[3 + 2 * n_out + 2 * o:5 + 2 * n_out + 2 * o]
                 for o in range(n_out)]
        sem_r = sc[3 + 4 * n_out:5 + 4 * n_out]
        sem_w = sc[5 + 4 * n_out:7 + 4 * n_out]
        sem_o = sc[7 + 4 * n_out:9 + 4 * n_out]
        base_g = _wid() * per_w
        pltpu.sync_copy(idx_hbm.at[pl.ds(base_g * gk, per_w * gk)], idx_all)

        def issue(t, b):
            pltpu.async_copy(
                table_hbm.at[idx_all.at[pl.ds(t * gk, gk)]],
                rows_v[b], sem_r[b])
            for o in range(n_out):
                pltpu.async_copy(
                    w_hbm[o].at[pl.ds((base_g + t) * gk, gk)],
                    w_v[o][b], sem_w[b])

        def wait_in(t, b):
            pltpu.make_async_copy(
                table_hbm.at[idx_all.at[pl.ds(t * gk, gk)]],
                rows_v[b], sem_r[b]).wait()
            for o in range(n_out):
                pltpu.make_async_copy(
                    w_hbm[o].at[pl.ds((base_g + t) * gk, gk)],
                    w_v[o][b], sem_w[b]).wait()

        def wait_out(t, b):
            for o in range(n_out):
                pltpu.make_async_copy(
                    out_v[o][b], out_hbm[o].at[pl.ds((base_g + t) * g, g)],
                    sem_o[b]).wait()

        issue(0, 0)

        def sstep(s, carry):
            for b in (0, 1):
                t = s * 2 + b

                @pl.when(t + 1 < per_w)
                def _():
                    issue(t + 1, 1 - b)

                wait_in(t, b)

                @pl.when(t >= 2)
                def _():
                    wait_out(t - 2, b)

                wv = [[w_v[o][b][gi * k + j] for gi in range(g)
                       for j in range(k_valid)] for o in range(n_out)]

                def chunk(ci, cc):
                    for u in range(4):
                        off = ci * 64 + u * 16
                        for gi in range(g):
                            loads = [rows_v[b][gi * k + j, pl.ds(off, 16)]
                                     for j in range(k_valid)]
                            for o in range(n_out):
                                acc = loads[0] * wv[o][gi * k_valid]
                                for j in range(1, k_valid):
                                    acc = acc + loads[j] * wv[o][gi * k_valid + j]
                                out_v[o][b][gi, pl.ds(off, 16)] = acc
                    return cc

                jax.lax.fori_loop(0, cw // 64, chunk, 0)
                for o in range(n_out):
                    pltpu.async_copy(
                        out_v[o][b], out_hbm[o].at[pl.ds((base_g + t) * g, g)],
                        sem_o[b])
            return carry

        jax.lax.fori_loop(0, per_w // 2, sstep, 0)
        for b in (0, 1):
            wait_out(per_w - 2 + b, b)

    res = kern(table, idx_flat, *w_list)
    return list(res) if isinstance(res, (list, tuple)) else [res]


def _pad_fanin(cols, vals_list, k_pad):
    nr, kk = cols.shape
    pc = jnp.concatenate(
        [cols, jnp.zeros((nr, k_pad - kk), jnp.int32)], axis=1).reshape(-1)
    pvs = [jnp.broadcast_to(
        jnp.concatenate([v, jnp.zeros((nr, k_pad - kk), jnp.float32)],
                        axis=1).reshape(-1)[:, None],
        (nr * k_pad, 16))
        for v in vals_list]
    return pc, pvs


def _stats_body(x_ref, o_ref):
    i = pl.program_id(0)

    @pl.when(i == 0)
    def _():
        o_ref[...] = jnp.zeros_like(o_ref)

    r = x_ref[...]
    o_ref[...] += jnp.stack([jnp.sum(r, axis=0), jnp.sum(r * r, axis=0)])


def _stats(rows):
    c = rows.shape[1]
    return pl.pallas_call(
        _stats_body,
        grid=(N_STAT_TILES,),
        in_specs=[pl.BlockSpec((ROW_TILE, c), lambda i: (i, 0))],
        out_specs=pl.BlockSpec((2, c), lambda i: (0, 0)),
        out_shape=jax.ShapeDtypeStruct((2, c), jnp.float32),
    )(rows)


def _stats3d_body(x_ref, o_ref):
    b = pl.program_id(0)
    j = pl.program_id(1)

    @pl.when((b == 0) & (j == 0))
    def _():
        o_ref[...] = jnp.zeros_like(o_ref)

    r = x_ref[0]
    rows = jax.lax.broadcasted_iota(jnp.int32, (VC_TILE, 1), 0)
    r = jnp.where(rows < NV_COARSE - j * VC_TILE, r, 0.0)
    o_ref[...] += jnp.stack([jnp.sum(r, axis=0), jnp.sum(r * r, axis=0)])


def _stats3d(arr):
    c = arr.shape[2]
    return pl.pallas_call(
        _stats3d_body,
        grid=(B, N_VC_TILES),
        in_specs=[pl.BlockSpec((1, VC_TILE, c), lambda b, j: (b, j, 0))],
        out_specs=pl.BlockSpec((2, c), lambda b, j: (0, 0)),
        out_shape=jax.ShapeDtypeStruct((2, c), jnp.float32),
    )(arr)


def _scale_off(sums, g, be):
    mean = sums[0] / N_ROWS
    var = sums[1] / N_ROWS - mean * mean
    scale = g * jax.lax.rsqrt(var + EPS)
    return scale, be - mean * scale


def _apply_gelu_body(x_ref, s_ref, g_ref, b_ref, o_ref):
    scale, off = _scale_off(s_ref[...], g_ref[...], b_ref[...])
    o_ref[...] = _gelu(x_ref[...] * scale[None, :] + off[None, :])


def _apply_gelu(rows, sums, g, be):
    c = rows.shape[1]
    return pl.pallas_call(
        _apply_gelu_body,
        grid=(N_APPLY_TILES,),
        in_specs=[
            pl.BlockSpec((ROW_TILE, c), lambda i: (i, 0)),
            pl.BlockSpec((2, c), lambda i: (0, 0)),
            pl.BlockSpec((c,), lambda i: (0,)),
            pl.BlockSpec((c,), lambda i: (0,)),
        ],
        out_specs=pl.BlockSpec((ROW_TILE, c), lambda i: (i, 0)),
        out_shape=jax.ShapeDtypeStruct((NPAD, c), jnp.float32),
    )(rows, sums, g, be)


def _feat_mm_body(h_ref, lap_ref, gve_ref, gvn_ref, w_ref, o_ref):
    ht = h_ref[...].reshape(VC_TILE * B, IN_CH)
    lt = lap_ref[...].reshape(VC_TILE * B, IN_CH)
    et = gve_ref[...].reshape(VC_TILE * B, IN_CH)
    nt = gvn_ref[...].reshape(VC_TILE * B, IN_CH)
    w = w_ref[...]
    acc = jnp.dot(ht, w[0:IN_CH], preferred_element_type=jnp.float32)
    acc += jnp.dot(lt, w[IN_CH:2 * IN_CH], preferred_element_type=jnp.float32)
    acc += jnp.dot(et, w[2 * IN_CH:3 * IN_CH], preferred_element_type=jnp.float32)
    acc += jnp.dot(nt, w[3 * IN_CH:4 * IN_CH], preferred_element_type=jnp.float32)
    o_ref[...] = acc.reshape(VC_TILE, B, IN_CH)


def _feat_mm(h3, lap3, gve3, gvn3, wm):
    spec = pl.BlockSpec((VC_TILE, B, IN_CH), lambda i: (i, 0, 0))
    return pl.pallas_call(
        _feat_mm_body,
        grid=(N_VC_TILES,),
        in_specs=[spec, spec, spec, spec,
                  pl.BlockSpec((4 * IN_CH, IN_CH), lambda i: (0, 0))],
        out_specs=spec,
        out_shape=jax.ShapeDtypeStruct((VPAD, B, IN_CH), jnp.float32),
    )(h3, lap3, gve3, gvn3, wm)


def _w3_body(m_ref, s_ref, g_ref, b_ref, w_ref, o_ref):
    scale, off = _scale_off(s_ref[...], g_ref[...], b_ref[...])
    z = _gelu(m_ref[...] * scale[None, None, :] + off[None, None, :])
    t = jnp.dot(z.reshape(VC_TILE * B, IN_CH), w_ref[...],
                preferred_element_type=jnp.float32)
    o_ref[...] = jnp.transpose(t.reshape(VC_TILE, B, OUT_CH), (1, 0, 2))


def _w3(m3, sums2, g2, be2, w3t):
    return pl.pallas_call(
        _w3_body,
        grid=(N_VC_TILES,),
        in_specs=[
            pl.BlockSpec((VC_TILE, B, IN_CH), lambda i: (i, 0, 0)),
            pl.BlockSpec((2, IN_CH), lambda i: (0, 0)),
            pl.BlockSpec((IN_CH,), lambda i: (0,)),
            pl.BlockSpec((IN_CH,), lambda i: (0,)),
            pl.BlockSpec((IN_CH, OUT_CH), lambda i: (0, 0)),
        ],
        out_specs=pl.BlockSpec((B, VC_TILE, OUT_CH), lambda i: (0, i, 0)),
        out_shape=jax.ShapeDtypeStruct((B, VPAD, OUT_CH), jnp.float32),
    )(m3, sums2, g2, be2, w3t)


def _final_body(t_ref, p_ref, s3_ref, g3_ref, b3_ref, ss_ref, gs_ref, bs_ref,
                o_ref):
    sc3, of3 = _scale_off(s3_ref[...], g3_ref[...], b3_ref[...])
    scs, ofs = _scale_off(ss_ref[...], gs_ref[...], bs_ref[...])
    r = (t_ref[0] * sc3[None, :] + of3[None, :]
         + p_ref[0] * scs[None, :] + ofs[None, :])
    r = _gelu(r)
    o_ref[0] = jnp.transpose(r, (1, 0))


def _final(t, ps, sums3, g3, be3, sums_s, gs, bes):
    vec = pl.BlockSpec((OUT_CH,), lambda b, i: (0,))
    st = pl.BlockSpec((2, OUT_CH), lambda b, i: (0, 0))
    blk = pl.BlockSpec((1, VC_TILE, OUT_CH), lambda b, i: (b, i, 0))
    return pl.pallas_call(
        _final_body,
        grid=(B, N_VC_TILES),
        in_specs=[blk, blk, st, vec, vec, st, vec, vec],
        out_specs=pl.BlockSpec((1, OUT_CH, VC_TILE), lambda b, i: (b, 0, i)),
        out_shape=jax.ShapeDtypeStruct((B, OUT_CH, NV_COARSE), jnp.float32),
    )(t, ps, sums3, g3, be3, sums_s, gs, bes)


def kernel(x, W1, b1, g1, be1, coeffs, mcb, g2, be2, W3, b3, g3, be3,
           Ws, bs, gs, bes, g_rows, g_cols, g_vals, l_rows, l_cols, l_vals,
           f_rows, f_cols, f_vals, ns, ew, vert_idx, patches):
    wcat_t = jnp.concatenate([W1, Ws], axis=0).T
    wm = jnp.transpose(coeffs, (2, 1, 0)).reshape(4 * IN_CH, IN_CH)
    w3t = W3.T
    pidx = vert_idx[patches].astype(jnp.int32)
    gc9 = jnp.transpose(g_cols.astype(jnp.int32).reshape(3, NF, 3),
                        (1, 0, 2)).reshape(NF, 9)
    gv9 = jnp.transpose(g_vals.reshape(3, NF, 3), (1, 0, 2)).reshape(NF, 3, 3)
    we9 = (ew[:, :, None] * gv9).reshape(NF, 9)
    wn9 = (ns[:, :, None] * gv9).reshape(NF, 9)
    lc7 = l_cols.astype(jnp.int32).reshape(NV_COARSE, 7)
    lv7 = l_vals.reshape(NV_COARSE, 7)
    fc6 = f_cols.astype(jnp.int32).reshape(NV_COARSE, 6)
    fv6 = f_vals.reshape(NV_COARSE, 6)

    pv8 = jnp.concatenate([pidx, pidx[:, :1]], axis=1)
    pv8 = jnp.concatenate(
        [pv8, jnp.full((VPAD - NV_COARSE, 8), ZROW, jnp.int32)], axis=0)
    boff = jnp.arange(B, dtype=jnp.int32) * VF_PAD
    pidx_h = (boff[None, :, None] + pv8[:, None, :]).reshape(-1)
    pidx_s = (boff[:, None, None] + pv8[None, :, :]).reshape(-1)

    rp = lambda a: jnp.concatenate(
        [a, jnp.zeros((VPAD - NV_COARSE,) + a.shape[1:], a.dtype)], axis=0)
    gc_f, (we_f, wn_f) = _pad_fanin(gc9, [we9, wn9], 12)
    lc_f, (lv_f,) = _pad_fanin(rp(lc7), [rp(lv7)], 8)
    fc_f, (fv_f,) = _pad_fanin(rp(fc6), [rp(fv6)], 8)

    y1, ys = _conv_fine(x, wcat_t)

    h_rows = _pool_sc(y1.reshape(B * VF_PAD, IN_CH), pidx_h, IN_CH)
    ps_rows = _pool_sc(ys.reshape(B * VF_PAD, OUT_CH), pidx_s, OUT_CH)
    ps = ps_rows.reshape(B, VPAD, OUT_CH)

    sums1 = _stats(h_rows)
    h = _apply_gelu(h_rows, sums1, g1, be1)
    h3 = h.reshape(VPAD, B, IN_CH)
    hv = h.reshape(VPAD, B * IN_CH)

    gfe, gfn = _spmm_sc(hv, gc_f, [we_f, wn_f], NF, 12, 2, 9)
    lap = _spmm_sc(hv, lc_f, [lv_f], VPAD, 8, 2, 7)[0]
    gve = _spmm_sc(gfe, fc_f, [fv_f], VPAD, 8, 2, 6)[0]
    gvn = _spmm_sc(gfn, fc_f, [fv_f], VPAD, 8, 2, 6)[0]
    lap = lap.reshape(VPAD, B, IN_CH)
    gve = gve.reshape(VPAD, B, IN_CH)
    gvn = gvn.reshape(VPAD, B, IN_CH)

    m3 = _feat_mm(h3, lap, gve, gvn, wm)

    sums2 = _stats(m3.reshape(NPAD, IN_CH))
    t = _w3(m3, sums2, g2, be2, w3t)

    sums3 = _stats3d(t)
    sums_s = _stats3d(ps)
    return _final(t, ps, sums3, g3, be3, sums_s, gs, bes)


# speedup vs baseline: 1.3510x; 1.1666x over previous
import functools
import jax
import jax.numpy as jnp
from jax.experimental import pallas as pl
from jax.experimental.pallas import tpu as pltpu
from jax.experimental.pallas import tpu_sc as plsc

B = 8
IN_CH = 256
OUT_CH = 512
NV_FINE = 10242
NV_COARSE = 2562
NF = 5120
VF_PAD = 10752
ZROW = 10242
VPAD = 2688
NPAD = B * VPAD
N_ROWS = B * NV_COARSE
ROW_TILE = 168
N_STAT_TILES = N_ROWS // ROW_TILE
N_APPLY_TILES = NPAD // ROW_TILE
VF_TILE = 512
N_VF_TILES = VF_PAD // VF_TILE
VC_TILE = 128
N_VC_TILES = VPAD // VC_TILE
EPS = 1e-5

_SC_INFO = plsc.get_sparse_core_info()
NWORK = _SC_INFO.num_cores * _SC_INFO.num_subcores


def _gelu(x):
    return 0.5 * x * (1.0 + jax.lax.erf(x * 0.7071067811865476))


def _wid():
    return (jax.lax.axis_index("s") * _SC_INFO.num_cores
            + jax.lax.axis_index("c"))


def _conv_fine_body(x_ref, w_ref, y1_ref, ys_ref):
    i = pl.program_id(1)
    xb = x_ref[0]
    y = jax.lax.dot_general(
        xb, w_ref[...], (((0,), (0,)), ((), ())),
        preferred_element_type=jnp.float32)
    rows = jax.lax.broadcasted_iota(jnp.int32, (VF_TILE, 1), 0)
    y = jnp.where(rows < NV_FINE - i * VF_TILE, y, 0.0)
    y1_ref[0] = y[:, :IN_CH]
    ys_ref[0] = y[:, IN_CH:]


def _conv_fine(x, wcat_t):
    return pl.pallas_call(
        _conv_fine_body,
        grid=(B, N_VF_TILES),
        in_specs=[
            pl.BlockSpec((1, IN_CH, VF_TILE), lambda b, i: (b, 0, i)),
            pl.BlockSpec((IN_CH, IN_CH + OUT_CH), lambda b, i: (0, 0)),
        ],
        out_specs=[
            pl.BlockSpec((1, VF_TILE, IN_CH), lambda b, i: (b, i, 0)),
            pl.BlockSpec((1, VF_TILE, OUT_CH), lambda b, i: (b, i, 0)),
        ],
        out_shape=[
            jax.ShapeDtypeStruct((B, VF_PAD, IN_CH), jnp.float32),
            jax.ShapeDtypeStruct((B, VF_PAD, OUT_CH), jnp.float32),
        ],
    )(x, wcat_t)


def _pool_sc(table, idx_flat, c):
    n_groups = VPAD
    per_w = n_groups // NWORK
    mesh = plsc.VectorSubcoreMesh(core_axis_name="c", subcore_axis_name="s")

    @functools.partial(
        pl.kernel, mesh=mesh,
        out_type=jax.ShapeDtypeStruct((n_groups * 8, c), jnp.float32),
        scratch_types=[
            pltpu.VMEM((per_w * 56,), jnp.int32),
            pltpu.VMEM((56, c), jnp.float32),
            pltpu.VMEM((56, c), jnp.float32),
            pltpu.VMEM((8, c), jnp.float32),
            pltpu.VMEM((8, c), jnp.float32),
            pltpu.SemaphoreType.DMA,
            pltpu.SemaphoreType.DMA,
            pltpu.SemaphoreType.DMA,
            pltpu.SemaphoreType.DMA,
        ],
    )
    def k(table_hbm, idx_hbm, out_hbm, idx_all, r0, r1, o0, o1,
          sr0, sr1, so0, so1):
        rows_v = [r0, r1]
        out_v = [o0, o1]
        sem_r = [sr0, sr1]
        sem_o = [so0, so1]
        base_g = _wid() * per_w
        pltpu.sync_copy(idx_hbm.at[pl.ds(base_g * 56, per_w * 56)], idx_all)

        def gather(t, b):
            return pltpu.async_copy(
                table_hbm.at[idx_all.at[pl.ds(t * 56, 56)]],
                rows_v[b], sem_r[b])

        gather(0, 0)

        def sstep(s, carry):
            for b in (0, 1):
                t = s * 2 + b

                @pl.when(t + 1 < per_w)
                def _():
                    gather(t + 1, 1 - b)

                pltpu.make_async_copy(
                    table_hbm.at[idx_all.at[pl.ds(t * 56, 56)]],
                    rows_v[b], sem_r[b]).wait()

                @pl.when(t >= 2)
                def _():
                    pltpu.make_async_copy(
                        out_v[b], out_hbm.at[pl.ds((base_g + t - 2) * 8, 8)],
                        sem_o[b]).wait()

                def chunk(ci, cc):
                    for u in range(4):
                        off = ci * 64 + u * 16
                        for gi in range(8):
                            acc = rows_v[b][gi * 7, pl.ds(off, 16)]
                            for j in range(1, 7):
                                acc = jnp.maximum(
                                    acc, rows_v[b][gi * 7 + j, pl.ds(off, 16)])
                            out_v[b][gi, pl.ds(off, 16)] = acc
                    return cc

                jax.lax.fori_loop(0, c // 64, chunk, 0)
                pltpu.async_copy(
                    out_v[b], out_hbm.at[pl.ds((base_g + t) * 8, 8)], sem_o[b])
            return carry

        jax.lax.fori_loop(0, per_w // 2, sstep, 0)
        for b in (0, 1):
            pltpu.make_async_copy(
                out_v[b], out_hbm.at[pl.ds((base_g + per_w - 2 + b) * 8, 8)],
                sem_o[b]).wait()

    return k(table, idx_flat)


def _spmm_sc(table, idx_flat, w_list, n_rows, k, g, k_valid):
    cw = table.shape[1]
    gk = g * k
    n_groups = n_rows // g
    per_w = n_groups // NWORK
    n_out = len(w_list)
    mesh = plsc.VectorSubcoreMesh(core_axis_name="c", subcore_axis_name="s")
    out_t = jax.ShapeDtypeStruct((n_rows, cw), jnp.float32)

    @functools.partial(
        pl.kernel, mesh=mesh,
        out_type=[out_t] * n_out,
        scratch_types=[pltpu.VMEM((per_w * gk,), jnp.int32)]
                      + [pltpu.VMEM((gk, cw), jnp.float32)] * 2
                      + [pltpu.VMEM((gk, 16), jnp.float32)] * (2 * n_out)
                      + [pltpu.VMEM((g, cw), jnp.float32)] * (2 * n_out)
                      + [pltpu.SemaphoreType.DMA] * 6,
    )
    def kern(*refs):
        table_hbm, idx_hbm = refs[0], refs[1]
        w_hbm = refs[2:2 + n_out]
        out_hbm = refs[2 + n_out:2 + 2 * n_out]
        sc = list(refs[2 + 2 * n_out:])
        idx_all = sc[0]
        rows_v = sc[1:3]
        w_v = [sc[3 + 2 * o:5 + 2 * o] for o in range(n_out)]
        out_v = [sc[3 + 2 * n_out + 2 * o:5 + 2 * n_out + 2 * o]
                 for o in range(n_out)]
        sem_r = sc[3 + 4 * n_out:5 + 4 * n_out]
        sem_w = sc[5 + 4 * n_out:7 + 4 * n_out]
        sem_o = sc[7 + 4 * n_out:9 + 4 * n_out]
        base_g = _wid() * per_w
        pltpu.sync_copy(idx_hbm.at[pl.ds(base_g * gk, per_w * gk)], idx_all)

        def issue(t, b):
            pltpu.async_copy(
                table_hbm.at[idx_all.at[pl.ds(t * gk, gk)]],
                rows_v[b], sem_r[b])
            for o in range(n_out):
                pltpu.async_copy(
                    w_hbm[o].at[pl.ds((base_g + t) * gk, gk)],
                    w_v[o][b], sem_w[b])

        def wait_in(t, b):
            pltpu.make_async_copy(
                table_hbm.at[idx_all.at[pl.ds(t * gk, gk)]],
                rows_v[b], sem_r[b]).wait()
            for o in range(n_out):
                pltpu.make_async_copy(
                    w_hbm[o].at[pl.ds((base_g + t) * gk, gk)],
                    w_v[o][b], sem_w[b]).wait()

        def wait_out(t, b):
            for o in range(n_out):
                pltpu.make_async_copy(
                    out_v[o][b], out_hbm[o].at[pl.ds((base_g + t) * g, g)],
                    sem_o[b]).wait()

        def body(t, b):
            @pl.when(t + 1 < per_w)
            def _():
                issue(t + 1, 1 - b)

            wait_in(t, b)

            @pl.when(t >= 2)
            def _():
                wait_out(t - 2, b)

            wv = [[w_v[o][b][gi * k + j] for gi in range(g)
                   for j in range(k_valid)] for o in range(n_out)]

            def chunk(ci, cc):
                for u in range(4):
                    off = ci * 64 + u * 16
                    for gi in range(g):
                        loads = [rows_v[b][gi * k + j, pl.ds(off, 16)]
                                 for j in range(k_valid)]
                        for o in range(n_out):
                            acc = loads[0] * wv[o][gi * k_valid]
                            for j in range(1, k_valid):
                                acc = acc + loads[j] * wv[o][gi * k_valid + j]
                            out_v[o][b][gi, pl.ds(off, 16)] = acc
                return cc

            jax.lax.fori_loop(0, cw // 64, chunk, 0)
            for o in range(n_out):
                pltpu.async_copy(
                    out_v[o][b], out_hbm[o].at[pl.ds((base_g + t) * g, g)],
                    sem_o[b])

        issue(0, 0)

        def sstep(s, carry):
            for b in (0, 1):
                body(s * 2 + b, b)
            return carry

        jax.lax.fori_loop(0, per_w // 2, sstep, 0)
        if per_w % 2:
            body(per_w - 1, 0)
        for t in (per_w - 2, per_w - 1):
            wait_out(t, t % 2)

    res = kern(table, idx_flat, *w_list)
    return list(res) if isinstance(res, (list, tuple)) else [res]


def _pad_fanin(cols, vals_list, k_pad):
    nr, kk = cols.shape
    pc = jnp.concatenate(
        [cols, jnp.zeros((nr, k_pad - kk), jnp.int32)], axis=1).reshape(-1)
    pvs = [jnp.broadcast_to(
        jnp.concatenate([v, jnp.zeros((nr, k_pad - kk), jnp.float32)],
                        axis=1).reshape(-1)[:, None],
        (nr * k_pad, 16))
        for v in vals_list]
    return pc, pvs


def _stats_body(x_ref, o_ref):
    i = pl.program_id(0)

    @pl.when(i == 0)
    def _():
        o_ref[...] = jnp.zeros_like(o_ref)

    r = x_ref[...]
    o_ref[...] += jnp.stack([jnp.sum(r, axis=0), jnp.sum(r * r, axis=0)])


def _stats(rows):
    c = rows.shape[1]
    return pl.pallas_call(
        _stats_body,
        grid=(N_STAT_TILES,),
        in_specs=[pl.BlockSpec((ROW_TILE, c), lambda i: (i, 0))],
        out_specs=pl.BlockSpec((2, c), lambda i: (0, 0)),
        out_shape=jax.ShapeDtypeStruct((2, c), jnp.float32),
    )(rows)


def _stats3d_body(x_ref, o_ref):
    b = pl.program_id(0)
    j = pl.program_id(1)

    @pl.when((b == 0) & (j == 0))
    def _():
        o_ref[...] = jnp.zeros_like(o_ref)

    r = x_ref[0]
    rows = jax.lax.broadcasted_iota(jnp.int32, (VC_TILE, 1), 0)
    r = jnp.where(rows < NV_COARSE - j * VC_TILE, r, 0.0)
    o_ref[...] += jnp.stack([jnp.sum(r, axis=0), jnp.sum(r * r, axis=0)])


def _stats3d(arr):
    c = arr.shape[2]
    return pl.pallas_call(
        _stats3d_body,
        grid=(B, N_VC_TILES),
        in_specs=[pl.BlockSpec((1, VC_TILE, c), lambda b, j: (b, j, 0))],
        out_specs=pl.BlockSpec((2, c), lambda b, j: (0, 0)),
        out_shape=jax.ShapeDtypeStruct((2, c), jnp.float32),
    )(arr)


def _scale_off(sums, g, be):
    mean = sums[0] / N_ROWS
    var = sums[1] / N_ROWS - mean * mean
    scale = g * jax.lax.rsqrt(var + EPS)
    return scale, be - mean * scale


def _apply_gelu_body(x_ref, s_ref, g_ref, b_ref, o_ref):
    scale, off = _scale_off(s_ref[...], g_ref[...], b_ref[...])
    o_ref[...] = _gelu(x_ref[...] * scale[None, :] + off[None, :])


def _apply_gelu(rows, sums, g, be):
    c = rows.shape[1]
    return pl.pallas_call(
        _apply_gelu_body,
        grid=(N_APPLY_TILES,),
        in_specs=[
            pl.BlockSpec((ROW_TILE, c), lambda i: (i, 0)),
            pl.BlockSpec((2, c), lambda i: (0, 0)),
            pl.BlockSpec((c,), lambda i: (0,)),
            pl.BlockSpec((c,), lambda i: (0,)),
        ],
        out_specs=pl.BlockSpec((ROW_TILE, c), lambda i: (i, 0)),
        out_shape=jax.ShapeDtypeStruct((NPAD, c), jnp.float32),
    )(rows, sums, g, be)


def _feat_mm_body(h_ref, lap_ref, gve_ref, gvn_ref, w_ref, o_ref):
    ht = h_ref[...].reshape(VC_TILE * B, IN_CH)
    lt = lap_ref[...].reshape(VC_TILE * B, IN_CH)
    et = gve_ref[...].reshape(VC_TILE * B, IN_CH)
    nt = gvn_ref[...].reshape(VC_TILE * B, IN_CH)
    w = w_ref[...]
    acc = jnp.dot(ht, w[0:IN_CH], preferred_element_type=jnp.float32)
    acc += jnp.dot(lt, w[IN_CH:2 * IN_CH], preferred_element_type=jnp.float32)
    acc += jnp.dot(et, w[2 * IN_CH:3 * IN_CH], preferred_element_type=jnp.float32)
    acc += jnp.dot(nt, w[3 * IN_CH:4 * IN_CH], preferred_element_type=jnp.float32)
    o_ref[...] = acc.reshape(VC_TILE, B, IN_CH)


def _feat_mm(h3, lap3, gve3, gvn3, wm):
    spec = pl.BlockSpec((VC_TILE, B, IN_CH), lambda i: (i, 0, 0))
    return pl.pallas_call(
        _feat_mm_body,
        grid=(N_VC_TILES,),
        in_specs=[spec, spec, spec, spec,
                  pl.BlockSpec((4 * IN_CH, IN_CH), lambda i: (0, 0))],
        out_specs=spec,
        out_shape=jax.ShapeDtypeStruct((VPAD, B, IN_CH), jnp.float32),
    )(h3, lap3, gve3, gvn3, wm)


def _w3_body(m_ref, s_ref, g_ref, b_ref, w_ref, o_ref):
    scale, off = _scale_off(s_ref[...], g_ref[...], b_ref[...])
    z = _gelu(m_ref[...] * scale[None, None, :] + off[None, None, :])
    t = jnp.dot(z.reshape(VC_TILE * B, IN_CH), w_ref[...],
                preferred_element_type=jnp.float32)
    o_ref[...] = jnp.transpose(t.reshape(VC_TILE, B, OUT_CH), (1, 0, 2))


def _w3(m3, sums2, g2, be2, w3t):
    return pl.pallas_call(
        _w3_body,
        grid=(N_VC_TILES,),
        in_specs=[
            pl.BlockSpec((VC_TILE, B, IN_CH), lambda i: (i, 0, 0)),
            pl.BlockSpec((2, IN_CH), lambda i: (0, 0)),
            pl.BlockSpec((IN_CH,), lambda i: (0,)),
            pl.BlockSpec((IN_CH,), lambda i: (0,)),
            pl.BlockSpec((IN_CH, OUT_CH), lambda i: (0, 0)),
        ],
        out_specs=pl.BlockSpec((B, VC_TILE, OUT_CH), lambda i: (0, i, 0)),
        out_shape=jax.ShapeDtypeStruct((B, VPAD, OUT_CH), jnp.float32),
    )(m3, sums2, g2, be2, w3t)


def _final_body(t_ref, p_ref, s3_ref, g3_ref, b3_ref, ss_ref, gs_ref, bs_ref,
                o_ref):
    sc3, of3 = _scale_off(s3_ref[...], g3_ref[...], b3_ref[...])
    scs, ofs = _scale_off(ss_ref[...], gs_ref[...], bs_ref[...])
    r = (t_ref[0] * sc3[None, :] + of3[None, :]
         + p_ref[0] * scs[None, :] + ofs[None, :])
    r = _gelu(r)
    o_ref[0] = jnp.transpose(r, (1, 0))


def _final(t, ps, sums3, g3, be3, sums_s, gs, bes):
    vec = pl.BlockSpec((OUT_CH,), lambda b, i: (0,))
    st = pl.BlockSpec((2, OUT_CH), lambda b, i: (0, 0))
    blk = pl.BlockSpec((1, VC_TILE, OUT_CH), lambda b, i: (b, i, 0))
    return pl.pallas_call(
        _final_body,
        grid=(B, N_VC_TILES),
        in_specs=[blk, blk, st, vec, vec, st, vec, vec],
        out_specs=pl.BlockSpec((1, OUT_CH, VC_TILE), lambda b, i: (b, 0, i)),
        out_shape=jax.ShapeDtypeStruct((B, OUT_CH, NV_COARSE), jnp.float32),
    )(t, ps, sums3, g3, be3, sums_s, gs, bes)


def kernel(x, W1, b1, g1, be1, coeffs, mcb, g2, be2, W3, b3, g3, be3,
           Ws, bs, gs, bes, g_rows, g_cols, g_vals, l_rows, l_cols, l_vals,
           f_rows, f_cols, f_vals, ns, ew, vert_idx, patches):
    wcat_t = jnp.concatenate([W1, Ws], axis=0).T
    wm = jnp.transpose(coeffs, (2, 1, 0)).reshape(4 * IN_CH, IN_CH)
    w3t = W3.T
    pidx = vert_idx[patches].astype(jnp.int32)
    gc9 = jnp.transpose(g_cols.astype(jnp.int32).reshape(3, NF, 3),
                        (1, 0, 2)).reshape(NF, 9)
    gv9 = jnp.transpose(g_vals.reshape(3, NF, 3), (1, 0, 2)).reshape(NF, 3, 3)
    we9 = (ew[:, :, None] * gv9).reshape(NF, 9)
    wn9 = (ns[:, :, None] * gv9).reshape(NF, 9)
    lc7 = l_cols.astype(jnp.int32).reshape(NV_COARSE, 7)
    lv7 = l_vals.reshape(NV_COARSE, 7)
    fc6 = f_cols.astype(jnp.int32).reshape(NV_COARSE, 6)
    fv6 = f_vals.reshape(NV_COARSE, 6)

    pv7 = jnp.concatenate(
        [pidx, jnp.full((VPAD - NV_COARSE, 7), ZROW, jnp.int32)], axis=0)
    boff = jnp.arange(B, dtype=jnp.int32) * VF_PAD
    pidx_h = (boff[None, :, None] + pv7[:, None, :]).reshape(-1)
    pidx_s = (boff[:, None, None] + pv7[None, :, :]).reshape(-1)

    rp = lambda a: jnp.concatenate(
        [a, jnp.zeros((VPAD - NV_COARSE,) + a.shape[1:], a.dtype)], axis=0)
    gc_f, (we_f, wn_f) = _pad_fanin(gc9, [we9, wn9], 12)
    lc_f, (lv_f,) = _pad_fanin(rp(lc7), [rp(lv7)], 8)
    fc_f, (fv_f,) = _pad_fanin(rp(fc6), [rp(fv6)], 6)

    y1, ys = _conv_fine(x, wcat_t)

    h_rows = _pool_sc(y1.reshape(B * VF_PAD, IN_CH), pidx_h, IN_CH)
    ps_rows = _pool_sc(ys.reshape(B * VF_PAD, OUT_CH), pidx_s, OUT_CH)
    ps = ps_rows.reshape(B, VPAD, OUT_CH)

    sums1 = _stats(h_rows)
    h = _apply_gelu(h_rows, sums1, g1, be1)
    h3 = h.reshape(VPAD, B, IN_CH)
    hv = h.reshape(VPAD, B * IN_CH)

    gfe, gfn = _spmm_sc(hv, gc_f, [we_f, wn_f], NF, 12, 2, 9)
    lap = _spmm_sc(hv, lc_f, [lv_f], VPAD, 8, 2, 7)[0]
    gve = _spmm_sc(gfe, fc_f, [fv_f], VPAD, 6, 4, 6)[0]
    gvn = _spmm_sc(gfn, fc_f, [fv_f], VPAD, 6, 4, 6)[0]
    lap = lap.reshape(VPAD, B, IN_CH)
    gve = gve.reshape(VPAD, B, IN_CH)
    gvn = gvn.reshape(VPAD, B, IN_CH)

    m3 = _feat_mm(h3, lap, gve, gvn, wm)

    sums2 = _stats(m3.reshape(NPAD, IN_CH))
    t = _w3(m3, sums2, g2, be2, w3t)

    sums3 = _stats3d(t)
    sums_s = _stats3d(ps)
    return _final(t, ps, sums3, g3, be3, sums_s, gs, bes)


# speedup vs baseline: 1.3536x; 1.0019x over previous
import functools
import jax
import jax.numpy as jnp
from jax.experimental import pallas as pl
from jax.experimental.pallas import tpu as pltpu
from jax.experimental.pallas import tpu_sc as plsc

B = 8
IN_CH = 256
OUT_CH = 512
NV_FINE = 10242
NV_COARSE = 2562
NF = 5120
VF_PAD = 10752
ZROW = 10242
VPAD = 2688
NPAD = B * VPAD
N_ROWS = B * NV_COARSE
ROW_TILE = 168
N_STAT_TILES = N_ROWS // ROW_TILE
N_APPLY_TILES = NPAD // ROW_TILE
VF_TILE = 512
N_VF_TILES = VF_PAD // VF_TILE
VC_TILE = 128
N_VC_TILES = VPAD // VC_TILE
EPS = 1e-5

_SC_INFO = plsc.get_sparse_core_info()
NWORK = _SC_INFO.num_cores * _SC_INFO.num_subcores


def _gelu(x):
    return 0.5 * x * (1.0 + jax.lax.erf(x * 0.7071067811865476))


def _wid():
    return (jax.lax.axis_index("s") * _SC_INFO.num_cores
            + jax.lax.axis_index("c"))


def _conv_fine_body(x_ref, w_ref, y1_ref, ys_ref):
    i = pl.program_id(1)
    xb = x_ref[0].astype(jnp.bfloat16)
    y = jax.lax.dot_general(
        xb, w_ref[...], (((0,), (0,)), ((), ())),
        preferred_element_type=jnp.float32)
    rows = jax.lax.broadcasted_iota(jnp.int32, (VF_TILE, 1), 0)
    y = jnp.where(rows < NV_FINE - i * VF_TILE, y, 0.0)
    y1_ref[0] = y[:, :IN_CH]
    ys_ref[0] = y[:, IN_CH:]


def _conv_fine(x, wcat_t):
    return pl.pallas_call(
        _conv_fine_body,
        grid=(B, N_VF_TILES),
        in_specs=[
            pl.BlockSpec((1, IN_CH, VF_TILE), lambda b, i: (b, 0, i)),
            pl.BlockSpec((IN_CH, IN_CH + OUT_CH), lambda b, i: (0, 0)),
        ],
        out_specs=[
            pl.BlockSpec((1, VF_TILE, IN_CH), lambda b, i: (b, i, 0)),
            pl.BlockSpec((1, VF_TILE, OUT_CH), lambda b, i: (b, i, 0)),
        ],
        out_shape=[
            jax.ShapeDtypeStruct((B, VF_PAD, IN_CH), jnp.float32),
            jax.ShapeDtypeStruct((B, VF_PAD, OUT_CH), jnp.float32),
        ],
    )(x, wcat_t)


def _pool_sc(table, idx_flat, c):
    n_groups = VPAD
    per_w = n_groups // NWORK
    mesh = plsc.VectorSubcoreMesh(core_axis_name="c", subcore_axis_name="s")

    @functools.partial(
        pl.kernel, mesh=mesh,
        out_type=jax.ShapeDtypeStruct((n_groups * 8, c), jnp.float32),
        scratch_types=[
            pltpu.VMEM((per_w * 56,), jnp.int32),
            pltpu.VMEM((56, c), jnp.float32),
            pltpu.VMEM((56, c), jnp.float32),
            pltpu.VMEM((8, c), jnp.float32),
            pltpu.VMEM((8, c), jnp.float32),
            pltpu.SemaphoreType.DMA,
            pltpu.SemaphoreType.DMA,
            pltpu.SemaphoreType.DMA,
            pltpu.SemaphoreType.DMA,
        ],
    )
    def k(table_hbm, idx_hbm, out_hbm, idx_all, r0, r1, o0, o1,
          sr0, sr1, so0, so1):
        rows_v = [r0, r1]
        out_v = [o0, o1]
        sem_r = [sr0, sr1]
        sem_o = [so0, so1]
        base_g = _wid() * per_w
        pltpu.sync_copy(idx_hbm.at[pl.ds(base_g * 56, per_w * 56)], idx_all)

        def gather(t, b):
            return pltpu.async_copy(
                table_hbm.at[idx_all.at[pl.ds(t * 56, 56)]],
                rows_v[b], sem_r[b])

        gather(0, 0)

        def sstep(s, carry):
            for b in (0, 1):
                t = s * 2 + b

                @pl.when(t + 1 < per_w)
                def _():
                    gather(t + 1, 1 - b)

                pltpu.make_async_copy(
                    table_hbm.at[idx_all.at[pl.ds(t * 56, 56)]],
                    rows_v[b], sem_r[b]).wait()

                @pl.when(t >= 2)
                def _():
                    pltpu.make_async_copy(
                        out_v[b], out_hbm.at[pl.ds((base_g + t - 2) * 8, 8)],
                        sem_o[b]).wait()

                def chunk(ci, cc):
                    for u in range(4):
                        off = ci * 64 + u * 16
                        for gi in range(8):
                            acc = rows_v[b][gi * 7, pl.ds(off, 16)]
                            for j in range(1, 7):
                                acc = jnp.maximum(
                                    acc, rows_v[b][gi * 7 + j, pl.ds(off, 16)])
                            out_v[b][gi, pl.ds(off, 16)] = acc
                    return cc

                jax.lax.fori_loop(0, c // 64, chunk, 0)
                pltpu.async_copy(
                    out_v[b], out_hbm.at[pl.ds((base_g + t) * 8, 8)], sem_o[b])
            return carry

        jax.lax.fori_loop(0, per_w // 2, sstep, 0)
        for b in (0, 1):
            pltpu.make_async_copy(
                out_v[b], out_hbm.at[pl.ds((base_g + per_w - 2 + b) * 8, 8)],
                sem_o[b]).wait()

    return k(table, idx_flat)


def _spmm_sc(table, idx_flat, w_list, n_rows, k, g, k_valid):
    cw = table.shape[1]
    gk = g * k
    n_groups = n_rows // g
    per_w = n_groups // NWORK
    n_out = len(w_list)
    mesh = plsc.VectorSubcoreMesh(core_axis_name="c", subcore_axis_name="s")
    out_t = jax.ShapeDtypeStruct((n_rows, cw), jnp.float32)

    @functools.partial(
        pl.kernel, mesh=mesh,
        out_type=[out_t] * n_out,
        scratch_types=[pltpu.VMEM((per_w * gk,), jnp.int32)]
                      + [pltpu.VMEM((gk, cw), jnp.float32)] * 2
                      + [pltpu.VMEM((gk, 16), jnp.float32)] * (2 * n_out)
                      + [pltpu.VMEM((g, cw), jnp.float32)] * (2 * n_out)
                      + [pltpu.SemaphoreType.DMA] * 6,
    )
    def kern(*refs):
        table_hbm, idx_hbm = refs[0], refs[1]
        w_hbm = refs[2:2 + n_out]
        out_hbm = refs[2 + n_out:2 + 2 * n_out]
        sc = list(refs[2 + 2 * n_out:])
        idx_all = sc[0]
        rows_v = sc[1:3]
        w_v = [sc[3 + 2 * o:5 + 2 * o] for o in range(n_out)]
        out_v = [sc[3 + 2 * n_out + 2 * o:5 + 2 * n_out + 2 * o]
                 for o in range(n_out)]
        sem_r = sc[3 + 4 * n_out:5 + 4 * n_out]
        sem_w = sc[5 + 4 * n_out:7 + 4 * n_out]
        sem_o = sc[7 + 4 * n_out:9 + 4 * n_out]
        base_g = _wid() * per_w
        pltpu.sync_copy(idx_hbm.at[pl.ds(base_g * gk, per_w * gk)], idx_all)

        def issue(t, b):
            pltpu.async_copy(
                table_hbm.at[idx_all.at[pl.ds(t * gk, gk)]],
                rows_v[b], sem_r[b])
            for o in range(n_out):
                pltpu.async_copy(
                    w_hbm[o].at[pl.ds((base_g + t) * gk, gk)],
                    w_v[o][b], sem_w[b])

        def wait_in(t, b):
            pltpu.make_async_copy(
                table_hbm.at[idx_all.at[pl.ds(t * gk, gk)]],
                rows_v[b], sem_r[b]).wait()
            for o in range(n_out):
                pltpu.make_async_copy(
                    w_hbm[o].at[pl.ds((base_g + t) * gk, gk)],
                    w_v[o][b], sem_w[b]).wait()

        def wait_out(t, b):
            for o in range(n_out):
                pltpu.make_async_copy(
                    out_v[o][b], out_hbm[o].at[pl.ds((base_g + t) * g, g)],
                    sem_o[b]).wait()

        def body(t, b):
            @pl.when(t + 1 < per_w)
            def _():
                issue(t + 1, 1 - b)

            wait_in(t, b)

            @pl.when(t >= 2)
            def _():
                wait_out(t - 2, b)

            wv = [[w_v[o][b][gi * k + j] for gi in range(g)
                   for j in range(k_valid)] for o in range(n_out)]

            def chunk(ci, cc):
                for u in range(4):
                    off = ci * 64 + u * 16
                    for gi in range(g):
                        loads = [rows_v[b][gi * k + j, pl.ds(off, 16)]
                                 for j in range(k_valid)]
                        for o in range(n_out):
                            acc = loads[0] * wv[o][gi * k_valid]
                            for j in range(1, k_valid):
                                acc = acc + loads[j] * wv[o][gi * k_valid + j]
                            out_v[o][b][gi, pl.ds(off, 16)] = acc
                return cc

            jax.lax.fori_loop(0, cw // 64, chunk, 0)
            for o in range(n_out):
                pltpu.async_copy(
                    out_v[o][b], out_hbm[o].at[pl.ds((base_g + t) * g, g)],
                    sem_o[b])

        issue(0, 0)

        def sstep(s, carry):
            for b in (0, 1):
                body(s * 2 + b, b)
            return carry

        jax.lax.fori_loop(0, per_w // 2, sstep, 0)
        if per_w % 2:
            body(per_w - 1, 0)
        for t in (per_w - 2, per_w - 1):
            wait_out(t, t % 2)

    res = kern(table, idx_flat, *w_list)
    return list(res) if isinstance(res, (list, tuple)) else [res]


def _pad_fanin(cols, vals_list, k_pad):
    nr, kk = cols.shape
    pc = jnp.concatenate(
        [cols, jnp.zeros((nr, k_pad - kk), jnp.int32)], axis=1).reshape(-1)
    pvs = [jnp.broadcast_to(
        jnp.concatenate([v, jnp.zeros((nr, k_pad - kk), jnp.float32)],
                        axis=1).reshape(-1)[:, None],
        (nr * k_pad, 16))
        for v in vals_list]
    return pc, pvs


def _stats_body(x_ref, o_ref):
    i = pl.program_id(0)

    @pl.when(i == 0)
    def _():
        o_ref[...] = jnp.zeros_like(o_ref)

    r = x_ref[...]
    o_ref[...] += jnp.stack([jnp.sum(r, axis=0), jnp.sum(r * r, axis=0)])


def _stats(rows):
    c = rows.shape[1]
    return pl.pallas_call(
        _stats_body,
        grid=(N_STAT_TILES,),
        in_specs=[pl.BlockSpec((ROW_TILE, c), lambda i: (i, 0))],
        out_specs=pl.BlockSpec((2, c), lambda i: (0, 0)),
        out_shape=jax.ShapeDtypeStruct((2, c), jnp.float32),
    )(rows)


def _stats3d_body(x_ref, o_ref):
    b = pl.program_id(0)
    j = pl.program_id(1)

    @pl.when((b == 0) & (j == 0))
    def _():
        o_ref[...] = jnp.zeros_like(o_ref)

    r = x_ref[0]
    rows = jax.lax.broadcasted_iota(jnp.int32, (VC_TILE, 1), 0)
    r = jnp.where(rows < NV_COARSE - j * VC_TILE, r, 0.0)
    o_ref[...] += jnp.stack([jnp.sum(r, axis=0), jnp.sum(r * r, axis=0)])


def _stats3d(arr):
    c = arr.shape[2]
    return pl.pallas_call(
        _stats3d_body,
        grid=(B, N_VC_TILES),
        in_specs=[pl.BlockSpec((1, VC_TILE, c), lambda b, j: (b, j, 0))],
        out_specs=pl.BlockSpec((2, c), lambda b, j: (0, 0)),
        out_shape=jax.ShapeDtypeStruct((2, c), jnp.float32),
    )(arr)


def _scale_off(sums, g, be):
    mean = sums[0] / N_ROWS
    var = sums[1] / N_ROWS - mean * mean
    scale = g * jax.lax.rsqrt(var + EPS)
    return scale, be - mean * scale


def _apply_gelu_body(x_ref, s_ref, g_ref, b_ref, o_ref):
    scale, off = _scale_off(s_ref[...], g_ref[...], b_ref[...])
    o_ref[...] = _gelu(x_ref[...] * scale[None, :] + off[None, :])


def _apply_gelu(rows, sums, g, be):
    c = rows.shape[1]
    return pl.pallas_call(
        _apply_gelu_body,
        grid=(N_APPLY_TILES,),
        in_specs=[
            pl.BlockSpec((ROW_TILE, c), lambda i: (i, 0)),
            pl.BlockSpec((2, c), lambda i: (0, 0)),
            pl.BlockSpec((c,), lambda i: (0,)),
            pl.BlockSpec((c,), lambda i: (0,)),
        ],
        out_specs=pl.BlockSpec((ROW_TILE, c), lambda i: (i, 0)),
        out_shape=jax.ShapeDtypeStruct((NPAD, c), jnp.float32),
    )(rows, sums, g, be)


def _feat_mm_body(h_ref, lap_ref, gve_ref, gvn_ref, w_ref, o_ref):
    ht = h_ref[...].reshape(VC_TILE * B, IN_CH).astype(jnp.bfloat16)
    lt = lap_ref[...].reshape(VC_TILE * B, IN_CH).astype(jnp.bfloat16)
    et = gve_ref[...].reshape(VC_TILE * B, IN_CH).astype(jnp.bfloat16)
    nt = gvn_ref[...].reshape(VC_TILE * B, IN_CH).astype(jnp.bfloat16)
    w = w_ref[...]
    acc = jnp.dot(ht, w[0:IN_CH], preferred_element_type=jnp.float32)
    acc += jnp.dot(lt, w[IN_CH:2 * IN_CH], preferred_element_type=jnp.float32)
    acc += jnp.dot(et, w[2 * IN_CH:3 * IN_CH], preferred_element_type=jnp.float32)
    acc += jnp.dot(nt, w[3 * IN_CH:4 * IN_CH], preferred_element_type=jnp.float32)
    o_ref[...] = acc.reshape(VC_TILE, B, IN_CH)


def _feat_mm(h3, lap3, gve3, gvn3, wm):
    spec = pl.BlockSpec((VC_TILE, B, IN_CH), lambda i: (i, 0, 0))
    return pl.pallas_call(
        _feat_mm_body,
        grid=(N_VC_TILES,),
        in_specs=[spec, spec, spec, spec,
                  pl.BlockSpec((4 * IN_CH, IN_CH), lambda i: (0, 0))],
        out_specs=spec,
        out_shape=jax.ShapeDtypeStruct((VPAD, B, IN_CH), jnp.float32),
    )(h3, lap3, gve3, gvn3, wm)


def _w3_body(m_ref, s_ref, g_ref, b_ref, w_ref, o_ref):
    scale, off = _scale_off(s_ref[...], g_ref[...], b_ref[...])
    z = _gelu(m_ref[...] * scale[None, None, :] + off[None, None, :])
    t = jnp.dot(z.reshape(VC_TILE * B, IN_CH).astype(jnp.bfloat16), w_ref[...],
                preferred_element_type=jnp.float32)
    o_ref[...] = jnp.transpose(t.reshape(VC_TILE, B, OUT_CH), (1, 0, 2))


def _w3(m3, sums2, g2, be2, w3t):
    return pl.pallas_call(
        _w3_body,
        grid=(N_VC_TILES,),
        in_specs=[
            pl.BlockSpec((VC_TILE, B, IN_CH), lambda i: (i, 0, 0)),
            pl.BlockSpec((2, IN_CH), lambda i: (0, 0)),
            pl.BlockSpec((IN_CH,), lambda i: (0,)),
            pl.BlockSpec((IN_CH,), lambda i: (0,)),
            pl.BlockSpec((IN_CH, OUT_CH), lambda i: (0, 0)),
        ],
        out_specs=pl.BlockSpec((B, VC_TILE, OUT_CH), lambda i: (0, i, 0)),
        out_shape=jax.ShapeDtypeStruct((B, VPAD, OUT_CH), jnp.float32),
    )(m3, sums2, g2, be2, w3t)


def _final_body(t_ref, p_ref, s3_ref, g3_ref, b3_ref, ss_ref, gs_ref, bs_ref,
                o_ref):
    sc3, of3 = _scale_off(s3_ref[...], g3_ref[...], b3_ref[...])
    scs, ofs = _scale_off(ss_ref[...], gs_ref[...], bs_ref[...])
    r = (t_ref[0] * sc3[None, :] + of3[None, :]
         + p_ref[0] * scs[None, :] + ofs[None, :])
    r = _gelu(r)
    o_ref[0] = jnp.transpose(r, (1, 0))


def _final(t, ps, sums3, g3, be3, sums_s, gs, bes):
    vec = pl.BlockSpec((OUT_CH,), lambda b, i: (0,))
    st = pl.BlockSpec((2, OUT_CH), lambda b, i: (0, 0))
    blk = pl.BlockSpec((1, VC_TILE, OUT_CH), lambda b, i: (b, i, 0))
    return pl.pallas_call(
        _final_body,
        grid=(B, N_VC_TILES),
        in_specs=[blk, blk, st, vec, vec, st, vec, vec],
        out_specs=pl.BlockSpec((1, OUT_CH, VC_TILE), lambda b, i: (b, 0, i)),
        out_shape=jax.ShapeDtypeStruct((B, OUT_CH, NV_COARSE), jnp.float32),
    )(t, ps, sums3, g3, be3, sums_s, gs, bes)


def kernel(x, W1, b1, g1, be1, coeffs, mcb, g2, be2, W3, b3, g3, be3,
           Ws, bs, gs, bes, g_rows, g_cols, g_vals, l_rows, l_cols, l_vals,
           f_rows, f_cols, f_vals, ns, ew, vert_idx, patches):
    wcat_t = jnp.concatenate([W1, Ws], axis=0).T.astype(jnp.bfloat16)
    wm = jnp.transpose(coeffs, (2, 1, 0)).reshape(
        4 * IN_CH, IN_CH).astype(jnp.bfloat16)
    w3t = W3.T.astype(jnp.bfloat16)
    pidx = vert_idx[patches].astype(jnp.int32)
    gc9 = jnp.transpose(g_cols.astype(jnp.int32).reshape(3, NF, 3),
                        (1, 0, 2)).reshape(NF, 9)
    gv9 = jnp.transpose(g_vals.reshape(3, NF, 3), (1, 0, 2)).reshape(NF, 3, 3)
    we9 = (ew[:, :, None] * gv9).reshape(NF, 9)
    wn9 = (ns[:, :, None] * gv9).reshape(NF, 9)
    lc7 = l_cols.astype(jnp.int32).reshape(NV_COARSE, 7)
    lv7 = l_vals.reshape(NV_COARSE, 7)
    fc6 = f_cols.astype(jnp.int32).reshape(NV_COARSE, 6)
    fv6 = f_vals.reshape(NV_COARSE, 6)

    pv7 = jnp.concatenate(
        [pidx, jnp.full((VPAD - NV_COARSE, 7), ZROW, jnp.int32)], axis=0)
    boff = jnp.arange(B, dtype=jnp.int32) * VF_PAD
    pidx_h = (boff[None, :, None] + pv7[:, None, :]).reshape(-1)
    pidx_s = (boff[:, None, None] + pv7[None, :, :]).reshape(-1)

    rp = lambda a: jnp.concatenate(
        [a, jnp.zeros((VPAD - NV_COARSE,) + a.shape[1:], a.dtype)], axis=0)
    gc_f, (we_f, wn_f) = _pad_fanin(gc9, [we9, wn9], 12)
    lc_f, (lv_f,) = _pad_fanin(rp(lc7), [rp(lv7)], 8)
    fc_f, (fv_f,) = _pad_fanin(rp(fc6), [rp(fv6)], 6)

    y1, ys = _conv_fine(x, wcat_t)

    h_rows = _pool_sc(y1.reshape(B * VF_PAD, IN_CH), pidx_h, IN_CH)
    ps_rows = _pool_sc(ys.reshape(B * VF_PAD, OUT_CH), pidx_s, OUT_CH)
    ps = ps_rows.reshape(B, VPAD, OUT_CH)

    sums1 = _stats(h_rows)
    h = _apply_gelu(h_rows, sums1, g1, be1)
    h3 = h.reshape(VPAD, B, IN_CH)
    hv = h.reshape(VPAD, B * IN_CH)

    gfe, gfn = _spmm_sc(hv, gc_f, [we_f, wn_f], NF, 12, 2, 9)
    lap = _spmm_sc(hv, lc_f, [lv_f], VPAD, 8, 2, 7)[0]
    gve = _spmm_sc(gfe, fc_f, [fv_f], VPAD, 6, 4, 6)[0]
    gvn = _spmm_sc(gfn, fc_f, [fv_f], VPAD, 6, 4, 6)[0]
    lap = lap.reshape(VPAD, B, IN_CH)
    gve = gve.reshape(VPAD, B, IN_CH)
    gvn = gvn.reshape(VPAD, B, IN_CH)

    m3 = _feat_mm(h3, lap, gve, gvn, wm)

    sums2 = _stats(m3.reshape(NPAD, IN_CH))
    t = _w3(m3, sums2, g2, be2, w3t)

    sums3 = _stats3d(t)
    sums_s = _stats3d(ps)
    return _final(t, ps, sums3, g3, be3, sums_s, gs, bes)
